# Initial kernel scaffold; baseline (speedup 1.0000x reference)
import jax
import jax.numpy as jnp
from jax.experimental import pallas as pl


def kernel(x, edge_index, BU_edge_index, root_index, batch, W1_td, b1_td, W2_td, b2_td, W1_bu, b1_bu, W2_bu, b2_bu, fc_W, fc_b):
    raise NotImplementedError("write your pallas kernel here")


# speedup vs baseline: 11.4723x; 11.4723x over previous
import functools

import jax
import jax.numpy as jnp
from jax import lax
from jax.experimental import pallas as pl
from jax.experimental.pallas import tpu as pltpu
from jax.experimental.pallas import tpu_sc as plsc

N = 10000
E = 320000
B = 64
F = 128

NCORE = 2
NSUB = 16
EPT = E // NSUB
K = 80
NCHUNK = EPT // K

RCH = 624
RLAST = N - 15 * RCH

_mesh = functools.partial(
    plsc.VectorSubcoreMesh, core_axis_name="c", subcore_axis_name="s")


@functools.partial(
    pl.kernel,
    mesh=_mesh(),
    out_type=jax.ShapeDtypeStruct((NCORE, N, F), jnp.float32),
    scratch_types=[
        pltpu.VMEM((K, F), jnp.float32),
        pltpu.VMEM((K,), jnp.int32),
        pltpu.VMEM_SHARED((N, F), jnp.float32),
    ],
)
def _sc_degree(dst_hbm, ones_hbm, zeros_hbm, out_hbm, ones_v, dst_v, deg_sp):
    c = lax.axis_index("c")
    s = lax.axis_index("s")

    row0 = pl.multiple_of(s * RCH, 8)

    @pl.when(s < 15)
    def _():
        pltpu.sync_copy(zeros_hbm.at[pl.ds(0, RCH)], deg_sp.at[pl.ds(row0, RCH)])

    @pl.when(s == 15)
    def _():
        pltpu.sync_copy(zeros_hbm, deg_sp.at[pl.ds(row0, RLAST)])

    pltpu.sync_copy(ones_hbm, ones_v)
    plsc.subcore_barrier()

    base = c * E + s * EPT

    def body(j, carry):
        off = pl.multiple_of(base + j * K, 8)
        pltpu.sync_copy(dst_hbm.at[pl.ds(off, K)], dst_v)
        pltpu.sync_copy(ones_v, deg_sp.at[dst_v], add=True)
        return carry

    lax.fori_loop(0, NCHUNK, body, 0)
    plsc.subcore_barrier()

    @pl.when(s < 15)
    def _():
        pltpu.sync_copy(deg_sp.at[pl.ds(row0, RCH)],
                        out_hbm.at[c, pl.ds(row0, RCH)])

    @pl.when(s == 15)
    def _():
        pltpu.sync_copy(deg_sp.at[pl.ds(row0, RLAST)],
                        out_hbm.at[c, pl.ds(row0, RLAST)])


@functools.partial(
    pl.kernel,
    mesh=_mesh(),
    out_type=jax.ShapeDtypeStruct((NCORE, N, F), jnp.float32),
    scratch_types=[
        pltpu.VMEM((K,), jnp.int32),
        pltpu.VMEM((K,), jnp.int32),
        pltpu.VMEM((K, F), jnp.float32),
        pltpu.VMEM_SHARED((N, F), jnp.float32),
        pltpu.SemaphoreType.DMA,
    ],
)
def _sc_scatter(tab_hbm, src_hbm, dst_hbm, zeros_hbm, out_hbm,
                src_v, dst_v, rows_v, acc_sp, sem):
    c = lax.axis_index("c")
    s = lax.axis_index("s")

    row0 = pl.multiple_of(s * RCH, 8)

    @pl.when(s < 15)
    def _():
        pltpu.sync_copy(zeros_hbm.at[pl.ds(0, RCH)], acc_sp.at[pl.ds(row0, RCH)])

    @pl.when(s == 15)
    def _():
        pltpu.sync_copy(zeros_hbm, acc_sp.at[pl.ds(row0, RLAST)])

    plsc.subcore_barrier()

    base = c * E + s * EPT

    def body(j, carry):
        off = pl.multiple_of(base + j * K, 8)
        pltpu.sync_copy(src_hbm.at[pl.ds(off, K)], src_v)
        pltpu.sync_copy(dst_hbm.at[pl.ds(off, K)], dst_v)
        pltpu.async_copy(tab_hbm.at[src_v], rows_v, sem).wait()
        pltpu.sync_copy(rows_v, acc_sp.at[dst_v], add=True)
        return carry

    lax.fori_loop(0, NCHUNK, body, 0)
    plsc.subcore_barrier()

    @pl.when(s < 15)
    def _():
        pltpu.sync_copy(acc_sp.at[pl.ds(row0, RCH)],
                        out_hbm.at[c, pl.ds(row0, RCH)])

    @pl.when(s == 15)
    def _():
        pltpu.sync_copy(acc_sp.at[pl.ds(row0, RLAST)],
                        out_hbm.at[c, pl.ds(row0, RLAST)])


BLK = 1000
GRID = N // BLK


def _tc_root_body(x_ref, root_ref, w2td_ref, w2bu_ref, rtd_ref, rbu_ref):
    col = lax.broadcasted_iota(jnp.int32, (B, N), 1)
    oh = (col == root_ref[...]).astype(jnp.float32)
    x_root = jnp.dot(oh, x_ref[...], preferred_element_type=jnp.float32)
    rx = jnp.maximum(x_root, 0.0)
    rtd_ref[...] = jnp.dot(rx, w2td_ref[F:, :],
                           preferred_element_type=jnp.float32)
    rbu_ref[...] = jnp.dot(rx, w2bu_ref[F:, :],
                           preferred_element_type=jnp.float32)


def _tc_root(x, root2, W2_td, W2_bu):
    return pl.pallas_call(
        _tc_root_body,
        out_shape=(
            jax.ShapeDtypeStruct((B, F), jnp.float32),
            jax.ShapeDtypeStruct((B, F), jnp.float32),
        ),
    )(x, root2, W2_td, W2_bu)


def _tc_xw1_body(x_ref, deg_ref, w1td_ref, w1bu_ref, out_ref):
    dinv_td = lax.rsqrt(deg_ref[0] + 1.0)
    dinv_bu = lax.rsqrt(deg_ref[1] + 1.0)
    xb = x_ref[...]
    out_ref[0] = jnp.dot(xb, w1td_ref[...],
                         preferred_element_type=jnp.float32) * dinv_td
    out_ref[1] = jnp.dot(xb, w1bu_ref[...],
                         preferred_element_type=jnp.float32) * dinv_bu


def _tc_xw1(x, degF, W1_td, W1_bu):
    return pl.pallas_call(
        _tc_xw1_body,
        grid=(GRID,),
        in_specs=[
            pl.BlockSpec((BLK, F), lambda i: (i, 0)),
            pl.BlockSpec((2, BLK, F), lambda i: (0, i, 0)),
            pl.BlockSpec((F, F), lambda i: (0, 0)),
            pl.BlockSpec((F, F), lambda i: (0, 0)),
        ],
        out_specs=pl.BlockSpec((2, BLK, F), lambda i: (0, i, 0)),
        out_shape=jax.ShapeDtypeStruct((2, N, F), jnp.float32),
    )(x, degF, W1_td, W1_bu)


def _tc_mid_body(acc1_ref, xw1_ref, deg_ref, batch_ref, root_ref,
                 rtd_ref, rbu_ref, w2td_ref, w2bu_ref, b1td_ref, b1bu_ref,
                 xw2_ref, x2root_ref):
    i = pl.program_id(0)
    ohb = (lax.broadcasted_iota(jnp.int32, (BLK, B), 1)
           == batch_ref[...]).astype(jnp.float32)
    rowsel = (lax.broadcasted_iota(jnp.int32, (B, BLK), 1) + i * BLK
              == root_ref[...]).astype(jnp.float32)

    def one_branch(k, w2_ref, r_ref, b1_ref):
        dinv = lax.rsqrt(deg_ref[k] + 1.0)
        h1 = dinv * (acc1_ref[k] + xw1_ref[k]) + b1_ref[...]
        rh = jnp.maximum(h1, 0.0)
        xw2 = (jnp.dot(rh, w2_ref[:F, :], preferred_element_type=jnp.float32)
               + jnp.dot(ohb, r_ref[...], preferred_element_type=jnp.float32)
               ) * dinv
        xw2_ref[k] = xw2
        part = jnp.dot(rowsel, h1, preferred_element_type=jnp.float32)
        return part

    ptd = one_branch(0, w2td_ref, rtd_ref, b1td_ref)
    pbu = one_branch(1, w2bu_ref, rbu_ref, b1bu_ref)

    @pl.when(i == 0)
    def _():
        x2root_ref[0] = ptd
        x2root_ref[1] = pbu

    @pl.when(i > 0)
    def _():
        x2root_ref[0] += ptd
        x2root_ref[1] += pbu


def _tc_mid(acc1, xw1, degF, batch2, root2, R_td, R_bu,
            W2_td, W2_bu, b1_td2, b1_bu2):
    return pl.pallas_call(
        _tc_mid_body,
        grid=(GRID,),
        in_specs=[
            pl.BlockSpec((2, BLK, F), lambda i: (0, i, 0)),
            pl.BlockSpec((2, BLK, F), lambda i: (0, i, 0)),
            pl.BlockSpec((2, BLK, F), lambda i: (0, i, 0)),
            pl.BlockSpec((BLK, 1), lambda i: (i, 0)),
            pl.BlockSpec((B, 1), lambda i: (0, 0)),
            pl.BlockSpec((B, F), lambda i: (0, 0)),
            pl.BlockSpec((B, F), lambda i: (0, 0)),
            pl.BlockSpec((2 * F, F), lambda i: (0, 0)),
            pl.BlockSpec((2 * F, F), lambda i: (0, 0)),
            pl.BlockSpec((1, F), lambda i: (0, 0)),
            pl.BlockSpec((1, F), lambda i: (0, 0)),
        ],
        out_specs=(
            pl.BlockSpec((2, BLK, F), lambda i: (0, i, 0)),
            pl.BlockSpec((2, B, F), lambda i: (0, 0, 0)),
        ),
        out_shape=(
            jax.ShapeDtypeStruct((2, N, F), jnp.float32),
            jax.ShapeDtypeStruct((2, B, F), jnp.float32),
        ),
    )(acc1, xw1, degF, batch2, root2, R_td, R_bu, W2_td, W2_bu,
      b1_td2, b1_bu2)


def _tc_final_body(acc2_ref, xw2_ref, deg_ref, batch_ref,
                   b2td_ref, b2bu_ref, x2root_ref, fcw_ref, fcb_ref,
                   out_ref, std_ref, sbu_ref, cnt_ref):
    i = pl.program_id(0)
    ohb = (lax.broadcasted_iota(jnp.int32, (BLK, B), 1)
           == batch_ref[...]).astype(jnp.float32)
    dn = (((0,), (0,)), ((), ()))

    def branch_p(k, b2_ref):
        dinv = lax.rsqrt(deg_ref[k] + 1.0)
        out2 = dinv * (acc2_ref[k] + xw2_ref[k]) + b2_ref[...]
        return jnp.maximum(out2, 0.0)

    ptd = branch_p(0, b2td_ref)
    pbu = branch_p(1, b2bu_ref)
    std = lax.dot_general(ohb, ptd, dn, preferred_element_type=jnp.float32)
    sbu = lax.dot_general(ohb, pbu, dn, preferred_element_type=jnp.float32)
    ones_col = jnp.ones((BLK, 1), jnp.float32)
    cnt = lax.dot_general(ohb, ones_col, dn,
                          preferred_element_type=jnp.float32)

    @pl.when(i == 0)
    def _():
        std_ref[...] = std
        sbu_ref[...] = sbu
        cnt_ref[...] = cnt

    @pl.when(i > 0)
    def _():
        std_ref[...] += std
        sbu_ref[...] += sbu
        cnt_ref[...] += cnt

    @pl.when(i == GRID - 1)
    def _():
        counts = cnt_ref[...]
        denom = jnp.maximum(counts, 1.0)
        nonempty = counts > 0.0
        p1td = std_ref[...] / denom
        p1bu = sbu_ref[...] / denom
        p2td = jnp.where(nonempty, x2root_ref[0], 0.0)
        p2bu = jnp.where(nonempty, x2root_ref[1], 0.0)
        hfc = jnp.concatenate([p1td, p2td, p1bu, p2bu], axis=1)
        logits = jnp.dot(hfc, fcw_ref[...],
                         preferred_element_type=jnp.float32) + fcb_ref[...]
        m = jnp.max(logits, axis=1, keepdims=True)
        lse = m + jnp.log(jnp.sum(jnp.exp(logits - m), axis=1, keepdims=True))
        out_ref[...] = logits - lse


def _tc_final(acc2, xw2, degF, batch2, b2_td2, b2_bu2, x2_root, fc_W, fc_b2):
    return pl.pallas_call(
        _tc_final_body,
        grid=(GRID,),
        in_specs=[
            pl.BlockSpec((2, BLK, F), lambda i: (0, i, 0)),
            pl.BlockSpec((2, BLK, F), lambda i: (0, i, 0)),
            pl.BlockSpec((2, BLK, F), lambda i: (0, i, 0)),
            pl.BlockSpec((BLK, 1), lambda i: (i, 0)),
            pl.BlockSpec((1, F), lambda i: (0, 0)),
            pl.BlockSpec((1, F), lambda i: (0, 0)),
            pl.BlockSpec((2, B, F), lambda i: (0, 0, 0)),
            pl.BlockSpec((4 * F, 2), lambda i: (0, 0)),
            pl.BlockSpec((1, 2), lambda i: (0, 0)),
        ],
        out_specs=pl.BlockSpec((B, 2), lambda i: (0, 0)),
        out_shape=jax.ShapeDtypeStruct((B, 2), jnp.float32),
        scratch_shapes=[
            pltpu.VMEM((B, F), jnp.float32),
            pltpu.VMEM((B, F), jnp.float32),
            pltpu.VMEM((B, 1), jnp.float32),
        ],
    )(acc2, xw2, degF, batch2, b2_td2, b2_bu2, x2_root, fc_W, fc_b2)


def kernel(x, edge_index, BU_edge_index, root_index, batch,
           W1_td, b1_td, W2_td, b2_td,
           W1_bu, b1_bu, W2_bu, b2_bu,
           fc_W, fc_b):
    src_flat = jnp.concatenate([edge_index[0], BU_edge_index[0] + N])
    dst_flat = jnp.concatenate([edge_index[1], BU_edge_index[1]])
    root2 = root_index.reshape(B, 1)
    batch2 = batch.reshape(N, 1)
    b1_td2 = b1_td.reshape(1, F)
    b1_bu2 = b1_bu.reshape(1, F)
    b2_td2 = b2_td.reshape(1, F)
    b2_bu2 = b2_bu.reshape(1, F)
    fc_b2 = fc_b.reshape(1, 2)

    ones_k = jnp.ones((K, F), jnp.float32)
    zeros_rows = jnp.zeros((RLAST, F), jnp.float32)

    degF = _sc_degree(dst_flat, ones_k, zeros_rows)

    R_td, R_bu = _tc_root(x, root2, W2_td, W2_bu)

    xw1 = _tc_xw1(x, degF, W1_td, W1_bu)
    acc1 = _sc_scatter(xw1.reshape(2 * N, F), src_flat, dst_flat, zeros_rows)

    xw2, x2_root = _tc_mid(acc1, xw1, degF, batch2, root2, R_td, R_bu,
                           W2_td, W2_bu, b1_td2, b1_bu2)
    acc2 = _sc_scatter(xw2.reshape(2 * N, F), src_flat, dst_flat, zeros_rows)

    return _tc_final(acc2, xw2, degF, batch2, b2_td2, b2_bu2,
                     x2_root, fc_W, fc_b2)


# speedup vs baseline: 27.6016x; 2.4059x over previous
import functools

import jax
import jax.numpy as jnp
from jax import lax
from jax.experimental import pallas as pl
from jax.experimental.pallas import tpu as pltpu
from jax.experimental.pallas import tpu_sc as plsc

N = 10000
E = 320000
B = 64
F = 128

NCORE = 2
NSUB = 16
EPT = E // NSUB
K = 40
NCHUNK = EPT // K
SEG = 10
NSEG = NCHUNK // SEG
SEGD = 50
NSEGD = NCHUNK // SEGD

RCH = 624
RLAST = N - 15 * RCH

_mesh = functools.partial(
    plsc.VectorSubcoreMesh, core_axis_name="c", subcore_axis_name="s")


NB = 4


@functools.partial(
    pl.kernel,
    mesh=_mesh(),
    out_type=jax.ShapeDtypeStruct((NCORE, N, F), jnp.float32),
    scratch_types=[
        pltpu.VMEM((K, F), jnp.float32),
        pltpu.VMEM((2, SEGD, K), jnp.int32),
        pltpu.VMEM_SHARED((N, F), jnp.float32),
    ] + [pltpu.SemaphoreType.DMA] * (NB + 2),
)
def _sc_degree(dst_hbm, ones_hbm, zeros_hbm, out_hbm, ones_v, didx, deg_sp,
               *sems):
    ssems, isems = sems[:NB], sems[NB:]
    c = lax.axis_index("c")
    s = lax.axis_index("s")

    row0 = pl.multiple_of(s * RCH, 8)

    @pl.when(s < 15)
    def _():
        pltpu.sync_copy(zeros_hbm.at[pl.ds(0, RCH)], deg_sp.at[pl.ds(row0, RCH)])

    @pl.when(s == 15)
    def _():
        pltpu.sync_copy(zeros_hbm, deg_sp.at[pl.ds(row0, RLAST)])

    pltpu.sync_copy(ones_hbm, ones_v)
    pltpu.sync_copy(dst_hbm.at[c, s, 0], didx.at[0])
    plsc.subcore_barrier()

    def body(g, carry):
        for sl in range(2 * SEGD):
            h, t = divmod(sl, SEGD)
            j = g * (2 * SEGD) + sl
            b = sl % NB

            if sl == 0:
                @pl.when(g > 0)
                def _():
                    pltpu.make_async_copy(dst_hbm.at[c, s, 0], didx.at[0],
                                          isems[0]).wait()

            @pl.when(j >= NB)
            def _():
                pltpu.make_async_copy(
                    ones_v, deg_sp.at[didx.at[0, 0]], ssems[b]).wait()

            if sl == NB:
                pltpu.async_copy(dst_hbm.at[c, s, 2 * g + 1], didx.at[1],
                                 isems[1])
            if sl == SEGD:
                pltpu.make_async_copy(dst_hbm.at[c, s, 0], didx.at[1],
                                      isems[1]).wait()
            if sl == SEGD + NB:
                @pl.when(g < NSEGD // 2 - 1)
                def _():
                    pltpu.async_copy(dst_hbm.at[c, s, 2 * g + 2], didx.at[0],
                                     isems[0])

            pltpu.async_copy(ones_v, deg_sp.at[didx.at[h, t]], ssems[b],
                             add=True)
        return carry

    lax.fori_loop(0, NSEGD // 2, body, 0)
    for i in range(NB):
        j = NCHUNK - NB + i
        pltpu.make_async_copy(ones_v, deg_sp.at[didx.at[0, 0]],
                              ssems[j % NB]).wait()
    plsc.subcore_barrier()

    @pl.when(s < 15)
    def _():
        pltpu.sync_copy(deg_sp.at[pl.ds(row0, RCH)],
                        out_hbm.at[c, pl.ds(row0, RCH)])

    @pl.when(s == 15)
    def _():
        pltpu.sync_copy(deg_sp.at[pl.ds(row0, RLAST)],
                        out_hbm.at[c, pl.ds(row0, RLAST)])


LAG = 2


@functools.partial(
    pl.kernel,
    mesh=_mesh(),
    out_type=jax.ShapeDtypeStruct((NCORE, N, F), jnp.float32),
    scratch_types=[
        pltpu.VMEM((2, 2, SEG, K), jnp.int32),
        pltpu.VMEM((NB, K, F), jnp.float32),
        pltpu.VMEM_SHARED((N, F), jnp.float32),
    ] + [pltpu.SemaphoreType.DMA] * (2 * NB + 2),
)
def _sc_scatter(tab_hbm, idx_hbm, zeros_hbm, out_hbm,
                cidx, rows_v, acc_sp, *sems):
    gsems, ssems, isems = sems[:NB], sems[NB:2 * NB], sems[2 * NB:]
    c = lax.axis_index("c")
    s = lax.axis_index("s")
    ahead = NB - LAG

    row0 = pl.multiple_of(s * RCH, 8)

    @pl.when(s < 15)
    def _():
        pltpu.sync_copy(zeros_hbm.at[pl.ds(0, RCH)], acc_sp.at[pl.ds(row0, RCH)])

    @pl.when(s == 15)
    def _():
        pltpu.sync_copy(zeros_hbm, acc_sp.at[pl.ds(row0, RLAST)])

    pltpu.sync_copy(idx_hbm.at[c, s, 0], cidx.at[0])
    plsc.subcore_barrier()

    for t in range(ahead):
        pltpu.async_copy(tab_hbm.at[cidx.at[0, 0, t]], rows_v.at[t], gsems[t])

    def body(g, carry):
        for sl in range(2 * SEG):
            h, t = divmod(sl, SEG)
            j = g * (2 * SEG) + sl
            b = sl % NB
            bb = (b + NB - LAG) % NB

            @pl.when(j >= LAG)
            def _():
                pltpu.make_async_copy(rows_v.at[bb],
                                      acc_sp.at[cidx.at[0, 1, 0]],
                                      ssems[bb]).wait()

            if sl == LAG + 1:
                pltpu.async_copy(idx_hbm.at[c, s, 2 * g + 1], cidx.at[1],
                                 isems[1])
            if sl == SEG + LAG + 1:
                @pl.when(g < NSEG // 2 - 1)
                def _():
                    pltpu.async_copy(idx_hbm.at[c, s, 2 * g + 2], cidx.at[0],
                                     isems[0])
            if sl == SEG - ahead:
                pltpu.make_async_copy(idx_hbm.at[c, s, 0], cidx.at[1],
                                      isems[1]).wait()
            if sl == 2 * SEG - ahead:
                @pl.when(g < NSEG // 2 - 1)
                def _():
                    pltpu.make_async_copy(idx_hbm.at[c, s, 0], cidx.at[0],
                                          isems[0]).wait()

            h3, t3 = divmod((sl + ahead) % (2 * SEG), SEG)

            @pl.when(j < NCHUNK - ahead)
            def _():
                pltpu.async_copy(tab_hbm.at[cidx.at[h3, 0, t3]],
                                 rows_v.at[bb], gsems[bb])

            pltpu.make_async_copy(tab_hbm.at[cidx.at[h, 0, t]],
                                  rows_v.at[b], gsems[b]).wait()
            pltpu.async_copy(rows_v.at[b], acc_sp.at[cidx.at[h, 1, t]],
                             ssems[b], add=True)
        return carry

    lax.fori_loop(0, NSEG // 2, body, 0)
    for i in range(LAG):
        j = NCHUNK - LAG + i
        pltpu.make_async_copy(rows_v.at[j % NB], acc_sp.at[cidx.at[0, 1, 0]],
                              ssems[j % NB]).wait()
    plsc.subcore_barrier()

    @pl.when(s < 15)
    def _():
        pltpu.sync_copy(acc_sp.at[pl.ds(row0, RCH)],
                        out_hbm.at[c, pl.ds(row0, RCH)])

    @pl.when(s == 15)
    def _():
        pltpu.sync_copy(acc_sp.at[pl.ds(row0, RLAST)],
                        out_hbm.at[c, pl.ds(row0, RLAST)])


BLK = 1000
GRID = N // BLK


def _tc_root_body(x_ref, root_ref, w2td_ref, w2bu_ref, rtd_ref, rbu_ref):
    col = lax.broadcasted_iota(jnp.int32, (B, N), 1)
    oh = (col == root_ref[...]).astype(jnp.float32)
    x_root = jnp.dot(oh, x_ref[...], preferred_element_type=jnp.float32)
    rx = jnp.maximum(x_root, 0.0)
    rtd_ref[...] = jnp.dot(rx, w2td_ref[F:, :],
                           preferred_element_type=jnp.float32)
    rbu_ref[...] = jnp.dot(rx, w2bu_ref[F:, :],
                           preferred_element_type=jnp.float32)


def _tc_root(x, root2, W2_td, W2_bu):
    return pl.pallas_call(
        _tc_root_body,
        out_shape=(
            jax.ShapeDtypeStruct((B, F), jnp.float32),
            jax.ShapeDtypeStruct((B, F), jnp.float32),
        ),
    )(x, root2, W2_td, W2_bu)


def _tc_xw1_body(x_ref, deg_ref, w1td_ref, w1bu_ref, out_ref):
    dinv_td = lax.rsqrt(deg_ref[0] + 1.0)
    dinv_bu = lax.rsqrt(deg_ref[1] + 1.0)
    xb = x_ref[...]
    out_ref[0] = jnp.dot(xb, w1td_ref[...],
                         preferred_element_type=jnp.float32) * dinv_td
    out_ref[1] = jnp.dot(xb, w1bu_ref[...],
                         preferred_element_type=jnp.float32) * dinv_bu


def _tc_xw1(x, degF, W1_td, W1_bu):
    return pl.pallas_call(
        _tc_xw1_body,
        grid=(GRID,),
        in_specs=[
            pl.BlockSpec((BLK, F), lambda i: (i, 0)),
            pl.BlockSpec((2, BLK, F), lambda i: (0, i, 0)),
            pl.BlockSpec((F, F), lambda i: (0, 0)),
            pl.BlockSpec((F, F), lambda i: (0, 0)),
        ],
        out_specs=pl.BlockSpec((2, BLK, F), lambda i: (0, i, 0)),
        out_shape=jax.ShapeDtypeStruct((2, N, F), jnp.float32),
    )(x, degF, W1_td, W1_bu)


def _tc_mid_body(acc1_ref, xw1_ref, deg_ref, batch_ref, root_ref,
                 rtd_ref, rbu_ref, w2td_ref, w2bu_ref, b1td_ref, b1bu_ref,
                 xw2_ref, x2root_ref):
    i = pl.program_id(0)
    ohb = (lax.broadcasted_iota(jnp.int32, (BLK, B), 1)
           == batch_ref[...]).astype(jnp.float32)
    rowsel = (lax.broadcasted_iota(jnp.int32, (B, BLK), 1) + i * BLK
              == root_ref[...]).astype(jnp.float32)

    def one_branch(k, w2_ref, r_ref, b1_ref):
        dinv = lax.rsqrt(deg_ref[k] + 1.0)
        h1 = dinv * (acc1_ref[k] + xw1_ref[k]) + b1_ref[...]
        rh = jnp.maximum(h1, 0.0)
        xw2 = (jnp.dot(rh, w2_ref[:F, :], preferred_element_type=jnp.float32)
               + jnp.dot(ohb, r_ref[...], preferred_element_type=jnp.float32)
               ) * dinv
        xw2_ref[k] = xw2
        part = jnp.dot(rowsel, h1, preferred_element_type=jnp.float32)
        return part

    ptd = one_branch(0, w2td_ref, rtd_ref, b1td_ref)
    pbu = one_branch(1, w2bu_ref, rbu_ref, b1bu_ref)

    @pl.when(i == 0)
    def _():
        x2root_ref[0] = ptd
        x2root_ref[1] = pbu

    @pl.when(i > 0)
    def _():
        x2root_ref[0] += ptd
        x2root_ref[1] += pbu


def _tc_mid(acc1, xw1, degF, batch2, root2, R_td, R_bu,
            W2_td, W2_bu, b1_td2, b1_bu2):
    return pl.pallas_call(
        _tc_mid_body,
        grid=(GRID,),
        in_specs=[
            pl.BlockSpec((2, BLK, F), lambda i: (0, i, 0)),
            pl.BlockSpec((2, BLK, F), lambda i: (0, i, 0)),
            pl.BlockSpec((2, BLK, F), lambda i: (0, i, 0)),
            pl.BlockSpec((BLK, 1), lambda i: (i, 0)),
            pl.BlockSpec((B, 1), lambda i: (0, 0)),
            pl.BlockSpec((B, F), lambda i: (0, 0)),
            pl.BlockSpec((B, F), lambda i: (0, 0)),
            pl.BlockSpec((2 * F, F), lambda i: (0, 0)),
            pl.BlockSpec((2 * F, F), lambda i: (0, 0)),
            pl.BlockSpec((1, F), lambda i: (0, 0)),
            pl.BlockSpec((1, F), lambda i: (0, 0)),
        ],
        out_specs=(
            pl.BlockSpec((2, BLK, F), lambda i: (0, i, 0)),
            pl.BlockSpec((2, B, F), lambda i: (0, 0, 0)),
        ),
        out_shape=(
            jax.ShapeDtypeStruct((2, N, F), jnp.float32),
            jax.ShapeDtypeStruct((2, B, F), jnp.float32),
        ),
    )(acc1, xw1, degF, batch2, root2, R_td, R_bu, W2_td, W2_bu,
      b1_td2, b1_bu2)


def _tc_final_body(acc2_ref, xw2_ref, deg_ref, batch_ref,
                   b2td_ref, b2bu_ref, x2root_ref, fcw_ref, fcb_ref,
                   out_ref, std_ref, sbu_ref, cnt_ref):
    i = pl.program_id(0)
    ohb = (lax.broadcasted_iota(jnp.int32, (BLK, B), 1)
           == batch_ref[...]).astype(jnp.float32)
    dn = (((0,), (0,)), ((), ()))

    def branch_p(k, b2_ref):
        dinv = lax.rsqrt(deg_ref[k] + 1.0)
        out2 = dinv * (acc2_ref[k] + xw2_ref[k]) + b2_ref[...]
        return jnp.maximum(out2, 0.0)

    ptd = branch_p(0, b2td_ref)
    pbu = branch_p(1, b2bu_ref)
    std = lax.dot_general(ohb, ptd, dn, preferred_element_type=jnp.float32)
    sbu = lax.dot_general(ohb, pbu, dn, preferred_element_type=jnp.float32)
    ones_col = jnp.ones((BLK, 1), jnp.float32)
    cnt = lax.dot_general(ohb, ones_col, dn,
                          preferred_element_type=jnp.float32)

    @pl.when(i == 0)
    def _():
        std_ref[...] = std
        sbu_ref[...] = sbu
        cnt_ref[...] = cnt

    @pl.when(i > 0)
    def _():
        std_ref[...] += std
        sbu_ref[...] += sbu
        cnt_ref[...] += cnt

    @pl.when(i == GRID - 1)
    def _():
        counts = cnt_ref[...]
        denom = jnp.maximum(counts, 1.0)
        nonempty = counts > 0.0
        p1td = std_ref[...] / denom
        p1bu = sbu_ref[...] / denom
        p2td = jnp.where(nonempty, x2root_ref[0], 0.0)
        p2bu = jnp.where(nonempty, x2root_ref[1], 0.0)
        hfc = jnp.concatenate([p1td, p2td, p1bu, p2bu], axis=1)
        logits = jnp.dot(hfc, fcw_ref[...],
                         preferred_element_type=jnp.float32) + fcb_ref[...]
        m = jnp.max(logits, axis=1, keepdims=True)
        lse = m + jnp.log(jnp.sum(jnp.exp(logits - m), axis=1, keepdims=True))
        out_ref[...] = logits - lse


def _tc_final(acc2, xw2, degF, batch2, b2_td2, b2_bu2, x2_root, fc_W, fc_b2):
    return pl.pallas_call(
        _tc_final_body,
        grid=(GRID,),
        in_specs=[
            pl.BlockSpec((2, BLK, F), lambda i: (0, i, 0)),
            pl.BlockSpec((2, BLK, F), lambda i: (0, i, 0)),
            pl.BlockSpec((2, BLK, F), lambda i: (0, i, 0)),
            pl.BlockSpec((BLK, 1), lambda i: (i, 0)),
            pl.BlockSpec((1, F), lambda i: (0, 0)),
            pl.BlockSpec((1, F), lambda i: (0, 0)),
            pl.BlockSpec((2, B, F), lambda i: (0, 0, 0)),
            pl.BlockSpec((4 * F, 2), lambda i: (0, 0)),
            pl.BlockSpec((1, 2), lambda i: (0, 0)),
        ],
        out_specs=pl.BlockSpec((B, 2), lambda i: (0, 0)),
        out_shape=jax.ShapeDtypeStruct((B, 2), jnp.float32),
        scratch_shapes=[
            pltpu.VMEM((B, F), jnp.float32),
            pltpu.VMEM((B, F), jnp.float32),
            pltpu.VMEM((B, 1), jnp.float32),
        ],
    )(acc2, xw2, degF, batch2, b2_td2, b2_bu2, x2_root, fc_W, fc_b2)


def kernel(x, edge_index, BU_edge_index, root_index, batch,
           W1_td, b1_td, W2_td, b2_td,
           W1_bu, b1_bu, W2_bu, b2_bu,
           fc_W, fc_b):
    src_flat = jnp.concatenate([edge_index[0], BU_edge_index[0] + N])
    dst_flat = jnp.concatenate([edge_index[1], BU_edge_index[1]])
    dstD = dst_flat.reshape(NCORE, NSUB, NSEGD, SEGD, K)
    cidx5 = jnp.stack(
        [src_flat.reshape(NCORE, NSUB, NSEG, SEG, K),
         dst_flat.reshape(NCORE, NSUB, NSEG, SEG, K)], axis=3)
    root2 = root_index.reshape(B, 1)
    batch2 = batch.reshape(N, 1)
    b1_td2 = b1_td.reshape(1, F)
    b1_bu2 = b1_bu.reshape(1, F)
    b2_td2 = b2_td.reshape(1, F)
    b2_bu2 = b2_bu.reshape(1, F)
    fc_b2 = fc_b.reshape(1, 2)

    ones_k = jnp.ones((K, F), jnp.float32)
    zeros_rows = jnp.zeros((RLAST, F), jnp.float32)

    degF = _sc_degree(dstD, ones_k, zeros_rows)

    R_td, R_bu = _tc_root(x, root2, W2_td, W2_bu)

    xw1 = _tc_xw1(x, degF, W1_td, W1_bu)
    acc1 = _sc_scatter(xw1.reshape(2 * N, F), cidx5, zeros_rows)

    xw2, x2_root = _tc_mid(acc1, xw1, degF, batch2, root2, R_td, R_bu,
                           W2_td, W2_bu, b1_td2, b1_bu2)
    acc2 = _sc_scatter(xw2.reshape(2 * N, F), cidx5, zeros_rows)

    return _tc_final(acc2, xw2, degF, batch2, b2_td2, b2_bu2,
                     x2_root, fc_W, fc_b2)


# speedup vs baseline: 27.8628x; 1.0095x over previous
import functools

import jax
import jax.numpy as jnp
from jax import lax
from jax.experimental import pallas as pl
from jax.experimental.pallas import tpu as pltpu
from jax.experimental.pallas import tpu_sc as plsc

N = 10000
E = 320000
B = 64
F = 128

NCORE = 2
NSUB = 16
EPT = E // NSUB
K = 40
NCHUNK = EPT // K
SEG = 10
NSEG = NCHUNK // SEG
KD = 80
NCHD = EPT // KD
SEGD = 25
NSEGD = NCHD // SEGD
NBD = 10

RCH = 624
RLAST = N - 15 * RCH

_mesh = functools.partial(
    plsc.VectorSubcoreMesh, core_axis_name="c", subcore_axis_name="s")


NB = 4


@functools.partial(
    pl.kernel,
    mesh=_mesh(),
    out_type=jax.ShapeDtypeStruct((NCORE, N, F), jnp.float32),
    scratch_types=[
        pltpu.VMEM((KD, F), jnp.float32),
        pltpu.VMEM((2, SEGD, KD), jnp.int32),
        pltpu.VMEM_SHARED((N, F), jnp.float32),
    ] + [pltpu.SemaphoreType.DMA] * (NBD + 2),
)
def _sc_degree(dst_hbm, ones_hbm, zeros_hbm, out_hbm, ones_v, didx, deg_sp,
               *sems):
    ssems, isems = sems[:NBD], sems[NBD:]
    c = lax.axis_index("c")
    s = lax.axis_index("s")

    row0 = pl.multiple_of(s * RCH, 8)

    @pl.when(s < 15)
    def _():
        pltpu.sync_copy(zeros_hbm.at[pl.ds(0, RCH)], deg_sp.at[pl.ds(row0, RCH)])

    @pl.when(s == 15)
    def _():
        pltpu.sync_copy(zeros_hbm, deg_sp.at[pl.ds(row0, RLAST)])

    pltpu.sync_copy(ones_hbm, ones_v)
    pltpu.sync_copy(dst_hbm.at[c, s, 0], didx.at[0])
    plsc.subcore_barrier()

    def body(g, carry):
        for sl in range(2 * SEGD):
            h, t = divmod(sl, SEGD)
            j = g * (2 * SEGD) + sl
            b = sl % NBD

            if sl == 0:
                @pl.when(g > 0)
                def _():
                    pltpu.make_async_copy(dst_hbm.at[c, s, 0], didx.at[0],
                                          isems[0]).wait()

            @pl.when(j >= NBD)
            def _():
                pltpu.make_async_copy(
                    ones_v, deg_sp.at[didx.at[0, 0]], ssems[b]).wait()

            if sl == NBD:
                pltpu.async_copy(dst_hbm.at[c, s, 2 * g + 1], didx.at[1],
                                 isems[1])
            if sl == SEGD:
                pltpu.make_async_copy(dst_hbm.at[c, s, 0], didx.at[1],
                                      isems[1]).wait()
            if sl == SEGD + NBD:
                @pl.when(g < NSEGD // 2 - 1)
                def _():
                    pltpu.async_copy(dst_hbm.at[c, s, 2 * g + 2], didx.at[0],
                                     isems[0])

            pltpu.async_copy(ones_v, deg_sp.at[didx.at[h, t]], ssems[b],
                             add=True)
        return carry

    lax.fori_loop(0, NSEGD // 2, body, 0)
    for i in range(NBD):
        j = NCHD - NBD + i
        pltpu.make_async_copy(ones_v, deg_sp.at[didx.at[0, 0]],
                              ssems[j % NBD]).wait()
    plsc.subcore_barrier()

    @pl.when(s < 15)
    def _():
        pltpu.sync_copy(deg_sp.at[pl.ds(row0, RCH)],
                        out_hbm.at[c, pl.ds(row0, RCH)])

    @pl.when(s == 15)
    def _():
        pltpu.sync_copy(deg_sp.at[pl.ds(row0, RLAST)],
                        out_hbm.at[c, pl.ds(row0, RLAST)])


LAG = 2


@functools.partial(
    pl.kernel,
    mesh=_mesh(),
    out_type=jax.ShapeDtypeStruct((NCORE, N, F), jnp.float32),
    scratch_types=[
        pltpu.VMEM((2, 2, SEG, K), jnp.int32),
        pltpu.VMEM((NB, K, F), jnp.float32),
        pltpu.VMEM_SHARED((N, F), jnp.float32),
    ] + [pltpu.SemaphoreType.DMA] * (2 * NB + 2),
)
def _sc_scatter(tab_hbm, idx_hbm, zeros_hbm, out_hbm,
                cidx, rows_v, acc_sp, *sems):
    gsems, ssems, isems = sems[:NB], sems[NB:2 * NB], sems[2 * NB:]
    c = lax.axis_index("c")
    s = lax.axis_index("s")
    ahead = NB - LAG

    row0 = pl.multiple_of(s * RCH, 8)

    @pl.when(s < 15)
    def _():
        pltpu.sync_copy(zeros_hbm.at[pl.ds(0, RCH)], acc_sp.at[pl.ds(row0, RCH)])

    @pl.when(s == 15)
    def _():
        pltpu.sync_copy(zeros_hbm, acc_sp.at[pl.ds(row0, RLAST)])

    pltpu.sync_copy(idx_hbm.at[c, s, 0], cidx.at[0])
    plsc.subcore_barrier()

    for t in range(ahead):
        pltpu.async_copy(tab_hbm.at[cidx.at[0, 0, t]], rows_v.at[t], gsems[t])

    def body(g, carry):
        for sl in range(2 * SEG):
            h, t = divmod(sl, SEG)
            j = g * (2 * SEG) + sl
            b = sl % NB
            bb = (b + NB - LAG) % NB

            @pl.when(j >= LAG)
            def _():
                pltpu.make_async_copy(rows_v.at[bb],
                                      acc_sp.at[cidx.at[0, 1, 0]],
                                      ssems[bb]).wait()

            if sl == LAG + 1:
                pltpu.async_copy(idx_hbm.at[c, s, 2 * g + 1], cidx.at[1],
                                 isems[1])
            if sl == SEG + LAG + 1:
                @pl.when(g < NSEG // 2 - 1)
                def _():
                    pltpu.async_copy(idx_hbm.at[c, s, 2 * g + 2], cidx.at[0],
                                     isems[0])
            if sl == SEG - ahead:
                pltpu.make_async_copy(idx_hbm.at[c, s, 0], cidx.at[1],
                                      isems[1]).wait()
            if sl == 2 * SEG - ahead:
                @pl.when(g < NSEG // 2 - 1)
                def _():
                    pltpu.make_async_copy(idx_hbm.at[c, s, 0], cidx.at[0],
                                          isems[0]).wait()

            h3, t3 = divmod((sl + ahead) % (2 * SEG), SEG)

            @pl.when(j < NCHUNK - ahead)
            def _():
                pltpu.async_copy(tab_hbm.at[cidx.at[h3, 0, t3]],
                                 rows_v.at[bb], gsems[bb])

            pltpu.make_async_copy(tab_hbm.at[cidx.at[h, 0, t]],
                                  rows_v.at[b], gsems[b]).wait()
            pltpu.async_copy(rows_v.at[b], acc_sp.at[cidx.at[h, 1, t]],
                             ssems[b], add=True)
        return carry

    lax.fori_loop(0, NSEG // 2, body, 0)
    for i in range(LAG):
        j = NCHUNK - LAG + i
        pltpu.make_async_copy(rows_v.at[j % NB], acc_sp.at[cidx.at[0, 1, 0]],
                              ssems[j % NB]).wait()
    plsc.subcore_barrier()

    @pl.when(s < 15)
    def _():
        pltpu.sync_copy(acc_sp.at[pl.ds(row0, RCH)],
                        out_hbm.at[c, pl.ds(row0, RCH)])

    @pl.when(s == 15)
    def _():
        pltpu.sync_copy(acc_sp.at[pl.ds(row0, RLAST)],
                        out_hbm.at[c, pl.ds(row0, RLAST)])


BLK = 1000
GRID = N // BLK


def _tc_xw1_body(x_ref, deg_ref, w1td_ref, w1bu_ref, root_ref,
                 w2td_ref, w2bu_ref, out_ref, rtd_ref, rbu_ref, xroot_ref):
    i = pl.program_id(0)
    dinv_td = lax.rsqrt(deg_ref[0] + 1.0)
    dinv_bu = lax.rsqrt(deg_ref[1] + 1.0)
    xb = x_ref[...]
    out_ref[0] = jnp.dot(xb, w1td_ref[...],
                         preferred_element_type=jnp.float32) * dinv_td
    out_ref[1] = jnp.dot(xb, w1bu_ref[...],
                         preferred_element_type=jnp.float32) * dinv_bu

    rowsel = (lax.broadcasted_iota(jnp.int32, (B, BLK), 1) + i * BLK
              == root_ref[...]).astype(jnp.float32)
    part = jnp.dot(rowsel, xb, preferred_element_type=jnp.float32)

    @pl.when(i == 0)
    def _():
        xroot_ref[...] = part

    @pl.when(i > 0)
    def _():
        xroot_ref[...] += part

    @pl.when(i == GRID - 1)
    def _():
        rx = jnp.maximum(xroot_ref[...], 0.0)
        rtd_ref[...] = jnp.dot(rx, w2td_ref[F:, :],
                               preferred_element_type=jnp.float32)
        rbu_ref[...] = jnp.dot(rx, w2bu_ref[F:, :],
                               preferred_element_type=jnp.float32)


def _tc_xw1(x, degF, W1_td, W1_bu, root2, W2_td, W2_bu):
    return pl.pallas_call(
        _tc_xw1_body,
        grid=(GRID,),
        in_specs=[
            pl.BlockSpec((BLK, F), lambda i: (i, 0)),
            pl.BlockSpec((2, BLK, F), lambda i: (0, i, 0)),
            pl.BlockSpec((F, F), lambda i: (0, 0)),
            pl.BlockSpec((F, F), lambda i: (0, 0)),
            pl.BlockSpec((B, 1), lambda i: (0, 0)),
            pl.BlockSpec((2 * F, F), lambda i: (0, 0)),
            pl.BlockSpec((2 * F, F), lambda i: (0, 0)),
        ],
        out_specs=(
            pl.BlockSpec((2, BLK, F), lambda i: (0, i, 0)),
            pl.BlockSpec((B, F), lambda i: (0, 0)),
            pl.BlockSpec((B, F), lambda i: (0, 0)),
        ),
        out_shape=(
            jax.ShapeDtypeStruct((2, N, F), jnp.float32),
            jax.ShapeDtypeStruct((B, F), jnp.float32),
            jax.ShapeDtypeStruct((B, F), jnp.float32),
        ),
        scratch_shapes=[pltpu.VMEM((B, F), jnp.float32)],
    )(x, degF, W1_td, W1_bu, root2, W2_td, W2_bu)


def _tc_mid_body(acc1_ref, xw1_ref, deg_ref, batch_ref, root_ref,
                 rtd_ref, rbu_ref, w2td_ref, w2bu_ref, b1td_ref, b1bu_ref,
                 xw2_ref, x2root_ref):
    i = pl.program_id(0)
    ohb = (lax.broadcasted_iota(jnp.int32, (BLK, B), 1)
           == batch_ref[...]).astype(jnp.float32)
    rowsel = (lax.broadcasted_iota(jnp.int32, (B, BLK), 1) + i * BLK
              == root_ref[...]).astype(jnp.float32)

    def one_branch(k, w2_ref, r_ref, b1_ref):
        dinv = lax.rsqrt(deg_ref[k] + 1.0)
        h1 = dinv * (acc1_ref[k] + xw1_ref[k]) + b1_ref[...]
        rh = jnp.maximum(h1, 0.0)
        xw2 = (jnp.dot(rh, w2_ref[:F, :], preferred_element_type=jnp.float32)
               + jnp.dot(ohb, r_ref[...], preferred_element_type=jnp.float32)
               ) * dinv
        xw2_ref[k] = xw2
        part = jnp.dot(rowsel, h1, preferred_element_type=jnp.float32)
        return part

    ptd = one_branch(0, w2td_ref, rtd_ref, b1td_ref)
    pbu = one_branch(1, w2bu_ref, rbu_ref, b1bu_ref)

    @pl.when(i == 0)
    def _():
        x2root_ref[0] = ptd
        x2root_ref[1] = pbu

    @pl.when(i > 0)
    def _():
        x2root_ref[0] += ptd
        x2root_ref[1] += pbu


def _tc_mid(acc1, xw1, degF, batch2, root2, R_td, R_bu,
            W2_td, W2_bu, b1_td2, b1_bu2):
    return pl.pallas_call(
        _tc_mid_body,
        grid=(GRID,),
        in_specs=[
            pl.BlockSpec((2, BLK, F), lambda i: (0, i, 0)),
            pl.BlockSpec((2, BLK, F), lambda i: (0, i, 0)),
            pl.BlockSpec((2, BLK, F), lambda i: (0, i, 0)),
            pl.BlockSpec((BLK, 1), lambda i: (i, 0)),
            pl.BlockSpec((B, 1), lambda i: (0, 0)),
            pl.BlockSpec((B, F), lambda i: (0, 0)),
            pl.BlockSpec((B, F), lambda i: (0, 0)),
            pl.BlockSpec((2 * F, F), lambda i: (0, 0)),
            pl.BlockSpec((2 * F, F), lambda i: (0, 0)),
            pl.BlockSpec((1, F), lambda i: (0, 0)),
            pl.BlockSpec((1, F), lambda i: (0, 0)),
        ],
        out_specs=(
            pl.BlockSpec((2, BLK, F), lambda i: (0, i, 0)),
            pl.BlockSpec((2, B, F), lambda i: (0, 0, 0)),
        ),
        out_shape=(
            jax.ShapeDtypeStruct((2, N, F), jnp.float32),
            jax.ShapeDtypeStruct((2, B, F), jnp.float32),
        ),
    )(acc1, xw1, degF, batch2, root2, R_td, R_bu, W2_td, W2_bu,
      b1_td2, b1_bu2)


def _tc_final_body(acc2_ref, xw2_ref, deg_ref, batch_ref,
                   b2td_ref, b2bu_ref, x2root_ref, fcw_ref, fcb_ref,
                   out_ref, std_ref, sbu_ref, cnt_ref):
    i = pl.program_id(0)
    ohb = (lax.broadcasted_iota(jnp.int32, (BLK, B), 1)
           == batch_ref[...]).astype(jnp.float32)
    dn = (((0,), (0,)), ((), ()))

    def branch_p(k, b2_ref):
        dinv = lax.rsqrt(deg_ref[k] + 1.0)
        out2 = dinv * (acc2_ref[k] + xw2_ref[k]) + b2_ref[...]
        return jnp.maximum(out2, 0.0)

    ptd = branch_p(0, b2td_ref)
    pbu = branch_p(1, b2bu_ref)
    std = lax.dot_general(ohb, ptd, dn, preferred_element_type=jnp.float32)
    sbu = lax.dot_general(ohb, pbu, dn, preferred_element_type=jnp.float32)
    ones_col = jnp.ones((BLK, 1), jnp.float32)
    cnt = lax.dot_general(ohb, ones_col, dn,
                          preferred_element_type=jnp.float32)

    @pl.when(i == 0)
    def _():
        std_ref[...] = std
        sbu_ref[...] = sbu
        cnt_ref[...] = cnt

    @pl.when(i > 0)
    def _():
        std_ref[...] += std
        sbu_ref[...] += sbu
        cnt_ref[...] += cnt

    @pl.when(i == GRID - 1)
    def _():
        counts = cnt_ref[...]
        denom = jnp.maximum(counts, 1.0)
        nonempty = counts > 0.0
        p1td = std_ref[...] / denom
        p1bu = sbu_ref[...] / denom
        p2td = jnp.where(nonempty, x2root_ref[0], 0.0)
        p2bu = jnp.where(nonempty, x2root_ref[1], 0.0)
        hfc = jnp.concatenate([p1td, p2td, p1bu, p2bu], axis=1)
        logits = jnp.dot(hfc, fcw_ref[...],
                         preferred_element_type=jnp.float32) + fcb_ref[...]
        m = jnp.max(logits, axis=1, keepdims=True)
        lse = m + jnp.log(jnp.sum(jnp.exp(logits - m), axis=1, keepdims=True))
        out_ref[...] = logits - lse


def _tc_final(acc2, xw2, degF, batch2, b2_td2, b2_bu2, x2_root, fc_W, fc_b2):
    return pl.pallas_call(
        _tc_final_body,
        grid=(GRID,),
        in_specs=[
            pl.BlockSpec((2, BLK, F), lambda i: (0, i, 0)),
            pl.BlockSpec((2, BLK, F), lambda i: (0, i, 0)),
            pl.BlockSpec((2, BLK, F), lambda i: (0, i, 0)),
            pl.BlockSpec((BLK, 1), lambda i: (i, 0)),
            pl.BlockSpec((1, F), lambda i: (0, 0)),
            pl.BlockSpec((1, F), lambda i: (0, 0)),
            pl.BlockSpec((2, B, F), lambda i: (0, 0, 0)),
            pl.BlockSpec((4 * F, 2), lambda i: (0, 0)),
            pl.BlockSpec((1, 2), lambda i: (0, 0)),
        ],
        out_specs=pl.BlockSpec((B, 2), lambda i: (0, 0)),
        out_shape=jax.ShapeDtypeStruct((B, 2), jnp.float32),
        scratch_shapes=[
            pltpu.VMEM((B, F), jnp.float32),
            pltpu.VMEM((B, F), jnp.float32),
            pltpu.VMEM((B, 1), jnp.float32),
        ],
    )(acc2, xw2, degF, batch2, b2_td2, b2_bu2, x2_root, fc_W, fc_b2)


def kernel(x, edge_index, BU_edge_index, root_index, batch,
           W1_td, b1_td, W2_td, b2_td,
           W1_bu, b1_bu, W2_bu, b2_bu,
           fc_W, fc_b):
    src_flat = jnp.concatenate([edge_index[0], BU_edge_index[0] + N])
    dst_flat = jnp.concatenate([edge_index[1], BU_edge_index[1]])
    dstD = dst_flat.reshape(NCORE, NSUB, NSEGD, SEGD, KD)
    cidx5 = jnp.stack(
        [src_flat.reshape(NCORE, NSUB, NSEG, SEG, K),
         dst_flat.reshape(NCORE, NSUB, NSEG, SEG, K)], axis=3)
    root2 = root_index.reshape(B, 1)
    batch2 = batch.reshape(N, 1)
    b1_td2 = b1_td.reshape(1, F)
    b1_bu2 = b1_bu.reshape(1, F)
    b2_td2 = b2_td.reshape(1, F)
    b2_bu2 = b2_bu.reshape(1, F)
    fc_b2 = fc_b.reshape(1, 2)

    ones_k = jnp.ones((KD, F), jnp.float32)
    zeros_rows = jnp.zeros((RLAST, F), jnp.float32)

    degF = _sc_degree(dstD, ones_k, zeros_rows)

    xw1, R_td, R_bu = _tc_xw1(x, degF, W1_td, W1_bu, root2, W2_td, W2_bu)
    acc1 = _sc_scatter(xw1.reshape(2 * N, F), cidx5, zeros_rows)

    xw2, x2_root = _tc_mid(acc1, xw1, degF, batch2, root2, R_td, R_bu,
                           W2_td, W2_bu, b1_td2, b1_bu2)
    acc2 = _sc_scatter(xw2.reshape(2 * N, F), cidx5, zeros_rows)

    return _tc_final(acc2, xw2, degF, batch2, b2_td2, b2_bu2,
                     x2_root, fc_W, fc_b2)


# speedup vs baseline: 32.5937x; 1.1698x over previous
import functools

import jax
import jax.numpy as jnp
from jax import lax
from jax.experimental import pallas as pl
from jax.experimental.pallas import tpu as pltpu
from jax.experimental.pallas import tpu_sc as plsc

N = 10000
E = 320000
B = 64
F = 128

NCORE = 2
NSUB = 16
EPT = E // NSUB
K = 40
NCHUNK = EPT // K
SEG = 10
NSEG = NCHUNK // SEG
LANES = 16
NPAD = 10112
RC = 640
RCL = NPAD - 15 * RC

RCH = 624
RLAST = N - 15 * RCH

_mesh = functools.partial(
    plsc.VectorSubcoreMesh, core_axis_name="c", subcore_axis_name="s")


NB = 4


@functools.partial(
    pl.kernel,
    mesh=_mesh(),
    compiler_params=pltpu.CompilerParams(needs_layout_passes=False),
    out_type=jax.ShapeDtypeStruct((NCORE, NPAD), jnp.float32),
    scratch_types=[
        pltpu.VMEM((NPAD,), jnp.float32),
        pltpu.VMEM((EPT,), jnp.int32),
        pltpu.VMEM((NSUB, RC), jnp.float32),
        pltpu.VMEM((RC,), jnp.float32),
        pltpu.VMEM_SHARED((NSUB, NPAD), jnp.float32),
    ],
)
def _sc_degree(dst_hbm, zeros_hbm, out_hbm, hist, didx, rbuf, sumv, stage):
    c = lax.axis_index("c")
    s = lax.axis_index("s")

    pltpu.sync_copy(zeros_hbm, hist)
    pltpu.sync_copy(dst_hbm.at[c, s], didx)
    ones16 = jnp.full((LANES,), 1.0, jnp.float32)

    def body(i, carry):
        idx = didx[pl.ds(i * LANES, LANES)]
        plsc.addupdate_scatter(hist, [idx], ones16)
        return carry

    lax.fori_loop(0, EPT // LANES, body, 0)
    pltpu.sync_copy(hist, stage.at[s])
    plsc.subcore_barrier()

    col0 = pl.multiple_of(s * RC, 8)

    @pl.when(s < 15)
    def _():
        pltpu.sync_copy(stage.at[:, pl.ds(col0, RC)], rbuf)

    @pl.when(s == 15)
    def _():
        pltpu.sync_copy(stage.at[:, pl.ds(col0, RCL)],
                        rbuf.at[:, pl.ds(0, RCL)])

    def msum(m, carry):
        acc = rbuf[0, pl.ds(m * LANES, LANES)]
        for r in range(1, NSUB):
            acc = acc + rbuf[r, pl.ds(m * LANES, LANES)]
        sumv[pl.ds(m * LANES, LANES)] = acc
        return carry

    lax.fori_loop(0, RC // LANES, msum, 0)

    @pl.when(s < 15)
    def _():
        pltpu.sync_copy(sumv, out_hbm.at[c, pl.ds(col0, RC)])

    @pl.when(s == 15)
    def _():
        pltpu.sync_copy(sumv.at[pl.ds(0, RCL)], out_hbm.at[c, pl.ds(col0, RCL)])


LAG = 2


@functools.partial(
    pl.kernel,
    mesh=_mesh(),
    out_type=jax.ShapeDtypeStruct((NCORE, N, F), jnp.float32),
    scratch_types=[
        pltpu.VMEM((2, 2, SEG, K), jnp.int32),
        pltpu.VMEM((NB, K, F), jnp.float32),
        pltpu.VMEM_SHARED((N, F), jnp.float32),
    ] + [pltpu.SemaphoreType.DMA] * (2 * NB + 2),
)
def _sc_scatter(tab_hbm, idx_hbm, zeros_hbm, out_hbm,
                cidx, rows_v, acc_sp, *sems):
    gsems, ssems, isems = sems[:NB], sems[NB:2 * NB], sems[2 * NB:]
    c = lax.axis_index("c")
    s = lax.axis_index("s")
    ahead = NB - LAG

    row0 = pl.multiple_of(s * RCH, 8)

    @pl.when(s < 15)
    def _():
        pltpu.sync_copy(zeros_hbm.at[pl.ds(0, RCH)], acc_sp.at[pl.ds(row0, RCH)])

    @pl.when(s == 15)
    def _():
        pltpu.sync_copy(zeros_hbm, acc_sp.at[pl.ds(row0, RLAST)])

    pltpu.sync_copy(idx_hbm.at[c, s, 0], cidx.at[0])
    plsc.subcore_barrier()

    for t in range(ahead):
        pltpu.async_copy(tab_hbm.at[cidx.at[0, 0, t]], rows_v.at[t], gsems[t])

    def body(g, carry):
        for sl in range(2 * SEG):
            h, t = divmod(sl, SEG)
            j = g * (2 * SEG) + sl
            b = sl % NB
            bb = (b + NB - LAG) % NB

            @pl.when(j >= LAG)
            def _():
                pltpu.make_async_copy(rows_v.at[bb],
                                      acc_sp.at[cidx.at[0, 1, 0]],
                                      ssems[bb]).wait()

            if sl == LAG + 1:
                pltpu.async_copy(idx_hbm.at[c, s, 2 * g + 1], cidx.at[1],
                                 isems[1])
            if sl == SEG + LAG + 1:
                @pl.when(g < NSEG // 2 - 1)
                def _():
                    pltpu.async_copy(idx_hbm.at[c, s, 2 * g + 2], cidx.at[0],
                                     isems[0])
            if sl == SEG - ahead:
                pltpu.make_async_copy(idx_hbm.at[c, s, 0], cidx.at[1],
                                      isems[1]).wait()
            if sl == 2 * SEG - ahead:
                @pl.when(g < NSEG // 2 - 1)
                def _():
                    pltpu.make_async_copy(idx_hbm.at[c, s, 0], cidx.at[0],
                                          isems[0]).wait()

            h3, t3 = divmod((sl + ahead) % (2 * SEG), SEG)

            @pl.when(j < NCHUNK - ahead)
            def _():
                pltpu.async_copy(tab_hbm.at[cidx.at[h3, 0, t3]],
                                 rows_v.at[bb], gsems[bb])

            pltpu.make_async_copy(tab_hbm.at[cidx.at[h, 0, t]],
                                  rows_v.at[b], gsems[b]).wait()
            pltpu.async_copy(rows_v.at[b], acc_sp.at[cidx.at[h, 1, t]],
                             ssems[b], add=True)
        return carry

    lax.fori_loop(0, NSEG // 2, body, 0)
    for i in range(LAG):
        j = NCHUNK - LAG + i
        pltpu.make_async_copy(rows_v.at[j % NB], acc_sp.at[cidx.at[0, 1, 0]],
                              ssems[j % NB]).wait()
    plsc.subcore_barrier()

    @pl.when(s < 15)
    def _():
        pltpu.sync_copy(acc_sp.at[pl.ds(row0, RCH)],
                        out_hbm.at[c, pl.ds(row0, RCH)])

    @pl.when(s == 15)
    def _():
        pltpu.sync_copy(acc_sp.at[pl.ds(row0, RLAST)],
                        out_hbm.at[c, pl.ds(row0, RLAST)])


BLK = 1000
GRID = N // BLK


def _tc_xw1_body(x_ref, deg_ref, w1td_ref, w1bu_ref, root_ref,
                 w2td_ref, w2bu_ref, out_ref, rtd_ref, rbu_ref, xroot_ref):
    i = pl.program_id(0)
    dinv_td = lax.rsqrt(deg_ref[:, 0:1] + 1.0)
    dinv_bu = lax.rsqrt(deg_ref[:, 1:2] + 1.0)
    xb = x_ref[...]
    out_ref[0] = jnp.dot(xb, w1td_ref[...],
                         preferred_element_type=jnp.float32) * dinv_td
    out_ref[1] = jnp.dot(xb, w1bu_ref[...],
                         preferred_element_type=jnp.float32) * dinv_bu

    rowsel = (lax.broadcasted_iota(jnp.int32, (B, BLK), 1) + i * BLK
              == root_ref[...]).astype(jnp.float32)
    part = jnp.dot(rowsel, xb, preferred_element_type=jnp.float32)

    @pl.when(i == 0)
    def _():
        xroot_ref[...] = part

    @pl.when(i > 0)
    def _():
        xroot_ref[...] += part

    @pl.when(i == GRID - 1)
    def _():
        rx = jnp.maximum(xroot_ref[...], 0.0)
        rtd_ref[...] = jnp.dot(rx, w2td_ref[F:, :],
                               preferred_element_type=jnp.float32)
        rbu_ref[...] = jnp.dot(rx, w2bu_ref[F:, :],
                               preferred_element_type=jnp.float32)


def _tc_xw1(x, degF, W1_td, W1_bu, root2, W2_td, W2_bu):
    return pl.pallas_call(
        _tc_xw1_body,
        grid=(GRID,),
        in_specs=[
            pl.BlockSpec((BLK, F), lambda i: (i, 0)),
            pl.BlockSpec((BLK, 2), lambda i: (i, 0)),
            pl.BlockSpec((F, F), lambda i: (0, 0)),
            pl.BlockSpec((F, F), lambda i: (0, 0)),
            pl.BlockSpec((B, 1), lambda i: (0, 0)),
            pl.BlockSpec((2 * F, F), lambda i: (0, 0)),
            pl.BlockSpec((2 * F, F), lambda i: (0, 0)),
        ],
        out_specs=(
            pl.BlockSpec((2, BLK, F), lambda i: (0, i, 0)),
            pl.BlockSpec((B, F), lambda i: (0, 0)),
            pl.BlockSpec((B, F), lambda i: (0, 0)),
        ),
        out_shape=(
            jax.ShapeDtypeStruct((2, N, F), jnp.float32),
            jax.ShapeDtypeStruct((B, F), jnp.float32),
            jax.ShapeDtypeStruct((B, F), jnp.float32),
        ),
        scratch_shapes=[pltpu.VMEM((B, F), jnp.float32)],
    )(x, degF, W1_td, W1_bu, root2, W2_td, W2_bu)


def _tc_mid_body(acc1_ref, xw1_ref, deg_ref, batch_ref, root_ref,
                 rtd_ref, rbu_ref, w2td_ref, w2bu_ref, b1td_ref, b1bu_ref,
                 xw2_ref, x2root_ref):
    i = pl.program_id(0)
    ohb = (lax.broadcasted_iota(jnp.int32, (BLK, B), 1)
           == batch_ref[...]).astype(jnp.float32)
    rowsel = (lax.broadcasted_iota(jnp.int32, (B, BLK), 1) + i * BLK
              == root_ref[...]).astype(jnp.float32)

    def one_branch(k, w2_ref, r_ref, b1_ref):
        dinv = lax.rsqrt(deg_ref[:, k:k + 1] + 1.0)
        h1 = dinv * (acc1_ref[k] + xw1_ref[k]) + b1_ref[...]
        rh = jnp.maximum(h1, 0.0)
        xw2 = (jnp.dot(rh, w2_ref[:F, :], preferred_element_type=jnp.float32)
               + jnp.dot(ohb, r_ref[...], preferred_element_type=jnp.float32)
               ) * dinv
        xw2_ref[k] = xw2
        part = jnp.dot(rowsel, h1, preferred_element_type=jnp.float32)
        return part

    ptd = one_branch(0, w2td_ref, rtd_ref, b1td_ref)
    pbu = one_branch(1, w2bu_ref, rbu_ref, b1bu_ref)

    @pl.when(i == 0)
    def _():
        x2root_ref[0] = ptd
        x2root_ref[1] = pbu

    @pl.when(i > 0)
    def _():
        x2root_ref[0] += ptd
        x2root_ref[1] += pbu


def _tc_mid(acc1, xw1, degF, batch2, root2, R_td, R_bu,
            W2_td, W2_bu, b1_td2, b1_bu2):
    return pl.pallas_call(
        _tc_mid_body,
        grid=(GRID,),
        in_specs=[
            pl.BlockSpec((2, BLK, F), lambda i: (0, i, 0)),
            pl.BlockSpec((2, BLK, F), lambda i: (0, i, 0)),
            pl.BlockSpec((BLK, 2), lambda i: (i, 0)),
            pl.BlockSpec((BLK, 1), lambda i: (i, 0)),
            pl.BlockSpec((B, 1), lambda i: (0, 0)),
            pl.BlockSpec((B, F), lambda i: (0, 0)),
            pl.BlockSpec((B, F), lambda i: (0, 0)),
            pl.BlockSpec((2 * F, F), lambda i: (0, 0)),
            pl.BlockSpec((2 * F, F), lambda i: (0, 0)),
            pl.BlockSpec((1, F), lambda i: (0, 0)),
            pl.BlockSpec((1, F), lambda i: (0, 0)),
        ],
        out_specs=(
            pl.BlockSpec((2, BLK, F), lambda i: (0, i, 0)),
            pl.BlockSpec((2, B, F), lambda i: (0, 0, 0)),
        ),
        out_shape=(
            jax.ShapeDtypeStruct((2, N, F), jnp.float32),
            jax.ShapeDtypeStruct((2, B, F), jnp.float32),
        ),
    )(acc1, xw1, degF, batch2, root2, R_td, R_bu, W2_td, W2_bu,
      b1_td2, b1_bu2)


def _tc_final_body(acc2_ref, xw2_ref, deg_ref, batch_ref,
                   b2td_ref, b2bu_ref, x2root_ref, fcw_ref, fcb_ref,
                   out_ref, std_ref, sbu_ref, cnt_ref):
    i = pl.program_id(0)
    ohb = (lax.broadcasted_iota(jnp.int32, (BLK, B), 1)
           == batch_ref[...]).astype(jnp.float32)
    dn = (((0,), (0,)), ((), ()))

    def branch_p(k, b2_ref):
        dinv = lax.rsqrt(deg_ref[:, k:k + 1] + 1.0)
        out2 = dinv * (acc2_ref[k] + xw2_ref[k]) + b2_ref[...]
        return jnp.maximum(out2, 0.0)

    ptd = branch_p(0, b2td_ref)
    pbu = branch_p(1, b2bu_ref)
    std = lax.dot_general(ohb, ptd, dn, preferred_element_type=jnp.float32)
    sbu = lax.dot_general(ohb, pbu, dn, preferred_element_type=jnp.float32)
    ones_col = jnp.ones((BLK, 1), jnp.float32)
    cnt = lax.dot_general(ohb, ones_col, dn,
                          preferred_element_type=jnp.float32)

    @pl.when(i == 0)
    def _():
        std_ref[...] = std
        sbu_ref[...] = sbu
        cnt_ref[...] = cnt

    @pl.when(i > 0)
    def _():
        std_ref[...] += std
        sbu_ref[...] += sbu
        cnt_ref[...] += cnt

    @pl.when(i == GRID - 1)
    def _():
        counts = cnt_ref[...]
        denom = jnp.maximum(counts, 1.0)
        nonempty = counts > 0.0
        p1td = std_ref[...] / denom
        p1bu = sbu_ref[...] / denom
        p2td = jnp.where(nonempty, x2root_ref[0], 0.0)
        p2bu = jnp.where(nonempty, x2root_ref[1], 0.0)
        hfc = jnp.concatenate([p1td, p2td, p1bu, p2bu], axis=1)
        logits = jnp.dot(hfc, fcw_ref[...],
                         preferred_element_type=jnp.float32) + fcb_ref[...]
        m = jnp.max(logits, axis=1, keepdims=True)
        lse = m + jnp.log(jnp.sum(jnp.exp(logits - m), axis=1, keepdims=True))
        out_ref[...] = logits - lse


def _tc_final(acc2, xw2, degF, batch2, b2_td2, b2_bu2, x2_root, fc_W, fc_b2):
    return pl.pallas_call(
        _tc_final_body,
        grid=(GRID,),
        in_specs=[
            pl.BlockSpec((2, BLK, F), lambda i: (0, i, 0)),
            pl.BlockSpec((2, BLK, F), lambda i: (0, i, 0)),
            pl.BlockSpec((BLK, 2), lambda i: (i, 0)),
            pl.BlockSpec((BLK, 1), lambda i: (i, 0)),
            pl.BlockSpec((1, F), lambda i: (0, 0)),
            pl.BlockSpec((1, F), lambda i: (0, 0)),
            pl.BlockSpec((2, B, F), lambda i: (0, 0, 0)),
            pl.BlockSpec((4 * F, 2), lambda i: (0, 0)),
            pl.BlockSpec((1, 2), lambda i: (0, 0)),
        ],
        out_specs=pl.BlockSpec((B, 2), lambda i: (0, 0)),
        out_shape=jax.ShapeDtypeStruct((B, 2), jnp.float32),
        scratch_shapes=[
            pltpu.VMEM((B, F), jnp.float32),
            pltpu.VMEM((B, F), jnp.float32),
            pltpu.VMEM((B, 1), jnp.float32),
        ],
    )(acc2, xw2, degF, batch2, b2_td2, b2_bu2, x2_root, fc_W, fc_b2)


def kernel(x, edge_index, BU_edge_index, root_index, batch,
           W1_td, b1_td, W2_td, b2_td,
           W1_bu, b1_bu, W2_bu, b2_bu,
           fc_W, fc_b):
    src_flat = jnp.concatenate([edge_index[0], BU_edge_index[0] + N])
    dst_flat = jnp.concatenate([edge_index[1], BU_edge_index[1]])
    dstD = dst_flat.reshape(NCORE, NSUB, EPT)
    cidx5 = jnp.stack(
        [src_flat.reshape(NCORE, NSUB, NSEG, SEG, K),
         dst_flat.reshape(NCORE, NSUB, NSEG, SEG, K)], axis=3)
    root2 = root_index.reshape(B, 1)
    batch2 = batch.reshape(N, 1)
    b1_td2 = b1_td.reshape(1, F)
    b1_bu2 = b1_bu.reshape(1, F)
    b2_td2 = b2_td.reshape(1, F)
    b2_bu2 = b2_bu.reshape(1, F)
    fc_b2 = fc_b.reshape(1, 2)

    zeros_n = jnp.zeros((NPAD,), jnp.float32)
    zeros_rows = jnp.zeros((RLAST, F), jnp.float32)

    deg2 = _sc_degree(dstD, zeros_n)
    degF = deg2[:, :N].T

    xw1, R_td, R_bu = _tc_xw1(x, degF, W1_td, W1_bu, root2, W2_td, W2_bu)
    acc1 = _sc_scatter(xw1.reshape(2 * N, F), cidx5, zeros_rows)

    xw2, x2_root = _tc_mid(acc1, xw1, degF, batch2, root2, R_td, R_bu,
                           W2_td, W2_bu, b1_td2, b1_bu2)
    acc2 = _sc_scatter(xw2.reshape(2 * N, F), cidx5, zeros_rows)

    return _tc_final(acc2, xw2, degF, batch2, b2_td2, b2_bu2,
                     x2_root, fc_W, fc_b2)


# speedup vs baseline: 32.8650x; 1.0083x over previous
import functools

import jax
import jax.numpy as jnp
from jax import lax
from jax.experimental import pallas as pl
from jax.experimental.pallas import tpu as pltpu
from jax.experimental.pallas import tpu_sc as plsc

N = 10000
E = 320000
B = 64
F = 128

NCORE = 2
NSUB = 16
EPT = E // NSUB
K = 100
NCHUNK = EPT // K
SEG = 10
NSEG = NCHUNK // SEG
LANES = 16
NPAD = 10112
RC = 640
RCL = NPAD - 15 * RC

RCH = 624
RLAST = N - 15 * RCH

_mesh = functools.partial(
    plsc.VectorSubcoreMesh, core_axis_name="c", subcore_axis_name="s")


NB = 2


@functools.partial(
    pl.kernel,
    mesh=_mesh(),
    compiler_params=pltpu.CompilerParams(needs_layout_passes=False),
    out_type=jax.ShapeDtypeStruct((NCORE, NPAD), jnp.float32),
    scratch_types=[
        pltpu.VMEM((NPAD,), jnp.float32),
        pltpu.VMEM((EPT,), jnp.int32),
        pltpu.VMEM((NSUB, RC), jnp.float32),
        pltpu.VMEM((RC,), jnp.float32),
        pltpu.VMEM_SHARED((NSUB, NPAD), jnp.float32),
    ],
)
def _sc_degree(dst_hbm, zeros_hbm, out_hbm, hist, didx, rbuf, sumv, stage):
    c = lax.axis_index("c")
    s = lax.axis_index("s")

    pltpu.sync_copy(zeros_hbm, hist)
    pltpu.sync_copy(dst_hbm.at[c, s], didx)
    ones16 = jnp.full((LANES,), 1.0, jnp.float32)

    def body(i, carry):
        idx = didx[pl.ds(i * LANES, LANES)]
        plsc.addupdate_scatter(hist, [idx], ones16)
        return carry

    lax.fori_loop(0, EPT // LANES, body, 0)
    pltpu.sync_copy(hist, stage.at[s])
    plsc.subcore_barrier()

    col0 = pl.multiple_of(s * RC, 8)

    @pl.when(s < 15)
    def _():
        pltpu.sync_copy(stage.at[:, pl.ds(col0, RC)], rbuf)

    @pl.when(s == 15)
    def _():
        pltpu.sync_copy(stage.at[:, pl.ds(col0, RCL)],
                        rbuf.at[:, pl.ds(0, RCL)])

    def msum(m, carry):
        acc = rbuf[0, pl.ds(m * LANES, LANES)]
        for r in range(1, NSUB):
            acc = acc + rbuf[r, pl.ds(m * LANES, LANES)]
        sumv[pl.ds(m * LANES, LANES)] = acc
        return carry

    lax.fori_loop(0, RC // LANES, msum, 0)

    @pl.when(s < 15)
    def _():
        pltpu.sync_copy(sumv, out_hbm.at[c, pl.ds(col0, RC)])

    @pl.when(s == 15)
    def _():
        pltpu.sync_copy(sumv.at[pl.ds(0, RCL)], out_hbm.at[c, pl.ds(col0, RCL)])


LAG = 1


@functools.partial(
    pl.kernel,
    mesh=_mesh(),
    out_type=jax.ShapeDtypeStruct((NCORE, N, F), jnp.float32),
    scratch_types=[
        pltpu.VMEM((2, 2, SEG, K), jnp.int32),
        pltpu.VMEM((NB, K, F), jnp.float32),
        pltpu.VMEM_SHARED((N, F), jnp.float32),
    ] + [pltpu.SemaphoreType.DMA] * (2 * NB + 2),
)
def _sc_scatter(tab_hbm, idx_hbm, zeros_hbm, out_hbm,
                cidx, rows_v, acc_sp, *sems):
    gsems, ssems, isems = sems[:NB], sems[NB:2 * NB], sems[2 * NB:]
    c = lax.axis_index("c")
    s = lax.axis_index("s")
    ahead = NB - LAG

    row0 = pl.multiple_of(s * RCH, 8)

    @pl.when(s < 15)
    def _():
        pltpu.sync_copy(zeros_hbm.at[pl.ds(0, RCH)], acc_sp.at[pl.ds(row0, RCH)])

    @pl.when(s == 15)
    def _():
        pltpu.sync_copy(zeros_hbm, acc_sp.at[pl.ds(row0, RLAST)])

    pltpu.sync_copy(idx_hbm.at[c, s, 0], cidx.at[0])
    plsc.subcore_barrier()

    for t in range(ahead):
        pltpu.async_copy(tab_hbm.at[cidx.at[0, 0, t]], rows_v.at[t], gsems[t])

    def body(g, carry):
        for sl in range(2 * SEG):
            h, t = divmod(sl, SEG)
            j = g * (2 * SEG) + sl
            b = sl % NB
            bb = (b + NB - LAG) % NB

            @pl.when(j >= LAG)
            def _():
                pltpu.make_async_copy(rows_v.at[bb],
                                      acc_sp.at[cidx.at[0, 1, 0]],
                                      ssems[bb]).wait()

            if sl == LAG + 1:
                pltpu.async_copy(idx_hbm.at[c, s, 2 * g + 1], cidx.at[1],
                                 isems[1])
            if sl == SEG + LAG + 1:
                @pl.when(g < NSEG // 2 - 1)
                def _():
                    pltpu.async_copy(idx_hbm.at[c, s, 2 * g + 2], cidx.at[0],
                                     isems[0])
            if sl == SEG - ahead:
                pltpu.make_async_copy(idx_hbm.at[c, s, 0], cidx.at[1],
                                      isems[1]).wait()
            if sl == 2 * SEG - ahead:
                @pl.when(g < NSEG // 2 - 1)
                def _():
                    pltpu.make_async_copy(idx_hbm.at[c, s, 0], cidx.at[0],
                                          isems[0]).wait()

            h3, t3 = divmod((sl + ahead) % (2 * SEG), SEG)

            @pl.when(j < NCHUNK - ahead)
            def _():
                pltpu.async_copy(tab_hbm.at[cidx.at[h3, 0, t3]],
                                 rows_v.at[bb], gsems[bb])

            pltpu.make_async_copy(tab_hbm.at[cidx.at[h, 0, t]],
                                  rows_v.at[b], gsems[b]).wait()
            pltpu.async_copy(rows_v.at[b], acc_sp.at[cidx.at[h, 1, t]],
                             ssems[b], add=True)
        return carry

    lax.fori_loop(0, NSEG // 2, body, 0)
    for i in range(LAG):
        j = NCHUNK - LAG + i
        pltpu.make_async_copy(rows_v.at[j % NB], acc_sp.at[cidx.at[0, 1, 0]],
                              ssems[j % NB]).wait()
    plsc.subcore_barrier()

    @pl.when(s < 15)
    def _():
        pltpu.sync_copy(acc_sp.at[pl.ds(row0, RCH)],
                        out_hbm.at[c, pl.ds(row0, RCH)])

    @pl.when(s == 15)
    def _():
        pltpu.sync_copy(acc_sp.at[pl.ds(row0, RLAST)],
                        out_hbm.at[c, pl.ds(row0, RLAST)])


BLK = 1000
GRID = N // BLK


def _tc_xw1_body(x_ref, deg_ref, w1td_ref, w1bu_ref, root_ref,
                 w2td_ref, w2bu_ref, out_ref, rtd_ref, rbu_ref, xroot_ref):
    i = pl.program_id(0)
    dinv_td = lax.rsqrt(deg_ref[:, 0:1] + 1.0)
    dinv_bu = lax.rsqrt(deg_ref[:, 1:2] + 1.0)
    xb = x_ref[...]
    out_ref[0] = jnp.dot(xb, w1td_ref[...],
                         preferred_element_type=jnp.float32) * dinv_td
    out_ref[1] = jnp.dot(xb, w1bu_ref[...],
                         preferred_element_type=jnp.float32) * dinv_bu

    rowsel = (lax.broadcasted_iota(jnp.int32, (B, BLK), 1) + i * BLK
              == root_ref[...]).astype(jnp.float32)
    part = jnp.dot(rowsel, xb, preferred_element_type=jnp.float32)

    @pl.when(i == 0)
    def _():
        xroot_ref[...] = part

    @pl.when(i > 0)
    def _():
        xroot_ref[...] += part

    @pl.when(i == GRID - 1)
    def _():
        rx = jnp.maximum(xroot_ref[...], 0.0)
        rtd_ref[...] = jnp.dot(rx, w2td_ref[F:, :],
                               preferred_element_type=jnp.float32)
        rbu_ref[...] = jnp.dot(rx, w2bu_ref[F:, :],
                               preferred_element_type=jnp.float32)


def _tc_xw1(x, degF, W1_td, W1_bu, root2, W2_td, W2_bu):
    return pl.pallas_call(
        _tc_xw1_body,
        grid=(GRID,),
        in_specs=[
            pl.BlockSpec((BLK, F), lambda i: (i, 0)),
            pl.BlockSpec((BLK, 2), lambda i: (i, 0)),
            pl.BlockSpec((F, F), lambda i: (0, 0)),
            pl.BlockSpec((F, F), lambda i: (0, 0)),
            pl.BlockSpec((B, 1), lambda i: (0, 0)),
            pl.BlockSpec((2 * F, F), lambda i: (0, 0)),
            pl.BlockSpec((2 * F, F), lambda i: (0, 0)),
        ],
        out_specs=(
            pl.BlockSpec((2, BLK, F), lambda i: (0, i, 0)),
            pl.BlockSpec((B, F), lambda i: (0, 0)),
            pl.BlockSpec((B, F), lambda i: (0, 0)),
        ),
        out_shape=(
            jax.ShapeDtypeStruct((2, N, F), jnp.float32),
            jax.ShapeDtypeStruct((B, F), jnp.float32),
            jax.ShapeDtypeStruct((B, F), jnp.float32),
        ),
        scratch_shapes=[pltpu.VMEM((B, F), jnp.float32)],
    )(x, degF, W1_td, W1_bu, root2, W2_td, W2_bu)


def _tc_mid_body(acc1_ref, xw1_ref, deg_ref, batch_ref, root_ref,
                 rtd_ref, rbu_ref, w2td_ref, w2bu_ref, b1td_ref, b1bu_ref,
                 xw2_ref, x2root_ref):
    i = pl.program_id(0)
    ohb = (lax.broadcasted_iota(jnp.int32, (BLK, B), 1)
           == batch_ref[...]).astype(jnp.float32)
    rowsel = (lax.broadcasted_iota(jnp.int32, (B, BLK), 1) + i * BLK
              == root_ref[...]).astype(jnp.float32)

    def one_branch(k, w2_ref, r_ref, b1_ref):
        dinv = lax.rsqrt(deg_ref[:, k:k + 1] + 1.0)
        h1 = dinv * (acc1_ref[k] + xw1_ref[k]) + b1_ref[...]
        rh = jnp.maximum(h1, 0.0)
        xw2 = (jnp.dot(rh, w2_ref[:F, :], preferred_element_type=jnp.float32)
               + jnp.dot(ohb, r_ref[...], preferred_element_type=jnp.float32)
               ) * dinv
        xw2_ref[k] = xw2
        part = jnp.dot(rowsel, h1, preferred_element_type=jnp.float32)
        return part

    ptd = one_branch(0, w2td_ref, rtd_ref, b1td_ref)
    pbu = one_branch(1, w2bu_ref, rbu_ref, b1bu_ref)

    @pl.when(i == 0)
    def _():
        x2root_ref[0] = ptd
        x2root_ref[1] = pbu

    @pl.when(i > 0)
    def _():
        x2root_ref[0] += ptd
        x2root_ref[1] += pbu


def _tc_mid(acc1, xw1, degF, batch2, root2, R_td, R_bu,
            W2_td, W2_bu, b1_td2, b1_bu2):
    return pl.pallas_call(
        _tc_mid_body,
        grid=(GRID,),
        in_specs=[
            pl.BlockSpec((2, BLK, F), lambda i: (0, i, 0)),
            pl.BlockSpec((2, BLK, F), lambda i: (0, i, 0)),
            pl.BlockSpec((BLK, 2), lambda i: (i, 0)),
            pl.BlockSpec((BLK, 1), lambda i: (i, 0)),
            pl.BlockSpec((B, 1), lambda i: (0, 0)),
            pl.BlockSpec((B, F), lambda i: (0, 0)),
            pl.BlockSpec((B, F), lambda i: (0, 0)),
            pl.BlockSpec((2 * F, F), lambda i: (0, 0)),
            pl.BlockSpec((2 * F, F), lambda i: (0, 0)),
            pl.BlockSpec((1, F), lambda i: (0, 0)),
            pl.BlockSpec((1, F), lambda i: (0, 0)),
        ],
        out_specs=(
            pl.BlockSpec((2, BLK, F), lambda i: (0, i, 0)),
            pl.BlockSpec((2, B, F), lambda i: (0, 0, 0)),
        ),
        out_shape=(
            jax.ShapeDtypeStruct((2, N, F), jnp.float32),
            jax.ShapeDtypeStruct((2, B, F), jnp.float32),
        ),
    )(acc1, xw1, degF, batch2, root2, R_td, R_bu, W2_td, W2_bu,
      b1_td2, b1_bu2)


def _tc_final_body(acc2_ref, xw2_ref, deg_ref, batch_ref,
                   b2td_ref, b2bu_ref, x2root_ref, fcw_ref, fcb_ref,
                   out_ref, std_ref, sbu_ref, cnt_ref):
    i = pl.program_id(0)
    ohb = (lax.broadcasted_iota(jnp.int32, (BLK, B), 1)
           == batch_ref[...]).astype(jnp.float32)
    dn = (((0,), (0,)), ((), ()))

    def branch_p(k, b2_ref):
        dinv = lax.rsqrt(deg_ref[:, k:k + 1] + 1.0)
        out2 = dinv * (acc2_ref[k] + xw2_ref[k]) + b2_ref[...]
        return jnp.maximum(out2, 0.0)

    ptd = branch_p(0, b2td_ref)
    pbu = branch_p(1, b2bu_ref)
    std = lax.dot_general(ohb, ptd, dn, preferred_element_type=jnp.float32)
    sbu = lax.dot_general(ohb, pbu, dn, preferred_element_type=jnp.float32)
    ones_col = jnp.ones((BLK, 1), jnp.float32)
    cnt = lax.dot_general(ohb, ones_col, dn,
                          preferred_element_type=jnp.float32)

    @pl.when(i == 0)
    def _():
        std_ref[...] = std
        sbu_ref[...] = sbu
        cnt_ref[...] = cnt

    @pl.when(i > 0)
    def _():
        std_ref[...] += std
        sbu_ref[...] += sbu
        cnt_ref[...] += cnt

    @pl.when(i == GRID - 1)
    def _():
        counts = cnt_ref[...]
        denom = jnp.maximum(counts, 1.0)
        nonempty = counts > 0.0
        p1td = std_ref[...] / denom
        p1bu = sbu_ref[...] / denom
        p2td = jnp.where(nonempty, x2root_ref[0], 0.0)
        p2bu = jnp.where(nonempty, x2root_ref[1], 0.0)
        hfc = jnp.concatenate([p1td, p2td, p1bu, p2bu], axis=1)
        logits = jnp.dot(hfc, fcw_ref[...],
                         preferred_element_type=jnp.float32) + fcb_ref[...]
        m = jnp.max(logits, axis=1, keepdims=True)
        lse = m + jnp.log(jnp.sum(jnp.exp(logits - m), axis=1, keepdims=True))
        out_ref[...] = logits - lse


def _tc_final(acc2, xw2, degF, batch2, b2_td2, b2_bu2, x2_root, fc_W, fc_b2):
    return pl.pallas_call(
        _tc_final_body,
        grid=(GRID,),
        in_specs=[
            pl.BlockSpec((2, BLK, F), lambda i: (0, i, 0)),
            pl.BlockSpec((2, BLK, F), lambda i: (0, i, 0)),
            pl.BlockSpec((BLK, 2), lambda i: (i, 0)),
            pl.BlockSpec((BLK, 1), lambda i: (i, 0)),
            pl.BlockSpec((1, F), lambda i: (0, 0)),
            pl.BlockSpec((1, F), lambda i: (0, 0)),
            pl.BlockSpec((2, B, F), lambda i: (0, 0, 0)),
            pl.BlockSpec((4 * F, 2), lambda i: (0, 0)),
            pl.BlockSpec((1, 2), lambda i: (0, 0)),
        ],
        out_specs=pl.BlockSpec((B, 2), lambda i: (0, 0)),
        out_shape=jax.ShapeDtypeStruct((B, 2), jnp.float32),
        scratch_shapes=[
            pltpu.VMEM((B, F), jnp.float32),
            pltpu.VMEM((B, F), jnp.float32),
            pltpu.VMEM((B, 1), jnp.float32),
        ],
    )(acc2, xw2, degF, batch2, b2_td2, b2_bu2, x2_root, fc_W, fc_b2)


def kernel(x, edge_index, BU_edge_index, root_index, batch,
           W1_td, b1_td, W2_td, b2_td,
           W1_bu, b1_bu, W2_bu, b2_bu,
           fc_W, fc_b):
    src_flat = jnp.concatenate([edge_index[0], BU_edge_index[0] + N])
    dst_flat = jnp.concatenate([edge_index[1], BU_edge_index[1]])
    dstD = dst_flat.reshape(NCORE, NSUB, EPT)
    cidx5 = jnp.stack(
        [src_flat.reshape(NCORE, NSUB, NSEG, SEG, K),
         dst_flat.reshape(NCORE, NSUB, NSEG, SEG, K)], axis=3)
    root2 = root_index.reshape(B, 1)
    batch2 = batch.reshape(N, 1)
    b1_td2 = b1_td.reshape(1, F)
    b1_bu2 = b1_bu.reshape(1, F)
    b2_td2 = b2_td.reshape(1, F)
    b2_bu2 = b2_bu.reshape(1, F)
    fc_b2 = fc_b.reshape(1, 2)

    zeros_n = jnp.zeros((NPAD,), jnp.float32)
    zeros_rows = jnp.zeros((RLAST, F), jnp.float32)

    deg2 = _sc_degree(dstD, zeros_n)
    degF = deg2[:, :N].T

    xw1, R_td, R_bu = _tc_xw1(x, degF, W1_td, W1_bu, root2, W2_td, W2_bu)
    acc1 = _sc_scatter(xw1.reshape(2 * N, F), cidx5, zeros_rows)

    xw2, x2_root = _tc_mid(acc1, xw1, degF, batch2, root2, R_td, R_bu,
                           W2_td, W2_bu, b1_td2, b1_bu2)
    acc2 = _sc_scatter(xw2.reshape(2 * N, F), cidx5, zeros_rows)

    return _tc_final(acc2, xw2, degF, batch2, b2_td2, b2_bu2,
                     x2_root, fc_W, fc_b2)


# speedup vs baseline: 34.2154x; 1.0411x over previous
import functools

import jax
import jax.numpy as jnp
from jax import lax
from jax.experimental import pallas as pl
from jax.experimental.pallas import tpu as pltpu
from jax.experimental.pallas import tpu_sc as plsc

N = 10000
E = 320000
B = 64
F = 128

NCORE = 2
NSUB = 16
EPT = E // NSUB
K = 100
NCHUNK = EPT // K
SEG = 10
NSEG = NCHUNK // SEG
LANES = 16
NPAD = 10112
RC = 640
RCL = NPAD - 15 * RC

RCH = 624
RLAST = N - 15 * RCH

_mesh = functools.partial(
    plsc.VectorSubcoreMesh, core_axis_name="c", subcore_axis_name="s")


NB = 2


@functools.partial(
    pl.kernel,
    mesh=_mesh(),
    compiler_params=pltpu.CompilerParams(needs_layout_passes=False),
    out_type=jax.ShapeDtypeStruct((NCORE, NPAD), jnp.float32),
    scratch_types=[
        pltpu.VMEM((NPAD,), jnp.float32),
        pltpu.VMEM((EPT,), jnp.int32),
        pltpu.VMEM((NSUB, RC), jnp.float32),
        pltpu.VMEM((RC,), jnp.float32),
        pltpu.VMEM_SHARED((NSUB, NPAD), jnp.float32),
    ],
)
def _sc_degree(dst_hbm, zeros_hbm, out_hbm, hist, didx, rbuf, sumv, stage):
    c = lax.axis_index("c")
    s = lax.axis_index("s")

    pltpu.sync_copy(zeros_hbm, hist)
    pltpu.sync_copy(dst_hbm.at[c, s], didx)
    ones16 = jnp.full((LANES,), 1.0, jnp.float32)

    def body(i, carry):
        idx = didx[pl.ds(i * LANES, LANES)]
        plsc.addupdate_scatter(hist, [idx], ones16)
        return carry

    lax.fori_loop(0, EPT // LANES, body, 0)
    pltpu.sync_copy(hist, stage.at[s])
    plsc.subcore_barrier()

    col0 = pl.multiple_of(s * RC, 8)

    @pl.when(s < 15)
    def _():
        pltpu.sync_copy(stage.at[:, pl.ds(col0, RC)], rbuf)

    @pl.when(s == 15)
    def _():
        pltpu.sync_copy(stage.at[:, pl.ds(col0, RCL)],
                        rbuf.at[:, pl.ds(0, RCL)])

    def msum(m, carry):
        acc = rbuf[0, pl.ds(m * LANES, LANES)]
        for r in range(1, NSUB):
            acc = acc + rbuf[r, pl.ds(m * LANES, LANES)]
        sumv[pl.ds(m * LANES, LANES)] = acc
        return carry

    lax.fori_loop(0, RC // LANES, msum, 0)

    @pl.when(s < 15)
    def _():
        pltpu.sync_copy(sumv, out_hbm.at[c, pl.ds(col0, RC)])

    @pl.when(s == 15)
    def _():
        pltpu.sync_copy(sumv.at[pl.ds(0, RCL)], out_hbm.at[c, pl.ds(col0, RCL)])


LAG = 1


@functools.partial(
    pl.kernel,
    mesh=_mesh(),
    out_type=jax.ShapeDtypeStruct((NCORE, N, F), jnp.float32),
    scratch_types=[
        pltpu.VMEM((2, 2, SEG, K), jnp.int32),
        pltpu.VMEM((NB, K, F), jnp.float32),
        pltpu.VMEM_SHARED((N, F), jnp.float32),
    ] + [pltpu.SemaphoreType.DMA] * (2 * NB + 2),
)
def _sc_scatter(tab_hbm, src_hbm, dst_hbm, zeros_hbm, out_hbm,
                cidx, rows_v, acc_sp, *sems):
    gsems, ssems, isems = sems[:NB], sems[NB:2 * NB], sems[2 * NB:]
    c = lax.axis_index("c")
    s = lax.axis_index("s")
    ahead = NB - LAG

    row0 = pl.multiple_of(s * RCH, 8)

    @pl.when(s < 15)
    def _():
        pltpu.sync_copy(zeros_hbm.at[pl.ds(0, RCH)], acc_sp.at[pl.ds(row0, RCH)])

    @pl.when(s == 15)
    def _():
        pltpu.sync_copy(zeros_hbm, acc_sp.at[pl.ds(row0, RLAST)])

    pltpu.sync_copy(src_hbm.at[c, s, 0], cidx.at[0, 0])
    pltpu.sync_copy(dst_hbm.at[c, s, 0], cidx.at[0, 1])
    tab_c = tab_hbm.at[c]
    plsc.subcore_barrier()

    for t in range(ahead):
        pltpu.async_copy(tab_c.at[cidx.at[0, 0, t]], rows_v.at[t], gsems[t])

    def body(g, carry):
        for sl in range(2 * SEG):
            h, t = divmod(sl, SEG)
            j = g * (2 * SEG) + sl
            b = sl % NB
            bb = (b + NB - LAG) % NB

            @pl.when(j >= LAG)
            def _():
                pltpu.make_async_copy(rows_v.at[bb],
                                      acc_sp.at[cidx.at[0, 1, 0]],
                                      ssems[bb]).wait()

            if sl == LAG + 1:
                pltpu.async_copy(src_hbm.at[c, s, 2 * g + 1], cidx.at[1, 0],
                                 isems[1])
                pltpu.async_copy(dst_hbm.at[c, s, 2 * g + 1], cidx.at[1, 1],
                                 isems[1])
            if sl == SEG + LAG + 1:
                @pl.when(g < NSEG // 2 - 1)
                def _():
                    pltpu.async_copy(src_hbm.at[c, s, 2 * g + 2], cidx.at[0, 0],
                                     isems[0])
                    pltpu.async_copy(dst_hbm.at[c, s, 2 * g + 2], cidx.at[0, 1],
                                     isems[0])
            if sl == SEG - ahead:
                pltpu.make_async_copy(src_hbm.at[c, s, 0], cidx.at[1, 0],
                                      isems[1]).wait()
                pltpu.make_async_copy(dst_hbm.at[c, s, 0], cidx.at[1, 1],
                                      isems[1]).wait()
            if sl == 2 * SEG - ahead:
                @pl.when(g < NSEG // 2 - 1)
                def _():
                    pltpu.make_async_copy(src_hbm.at[c, s, 0], cidx.at[0, 0],
                                          isems[0]).wait()
                    pltpu.make_async_copy(dst_hbm.at[c, s, 0], cidx.at[0, 1],
                                          isems[0]).wait()

            h3, t3 = divmod((sl + ahead) % (2 * SEG), SEG)

            @pl.when(j < NCHUNK - ahead)
            def _():
                pltpu.async_copy(tab_c.at[cidx.at[h3, 0, t3]],
                                 rows_v.at[bb], gsems[bb])

            pltpu.make_async_copy(tab_c.at[cidx.at[h, 0, t]],
                                  rows_v.at[b], gsems[b]).wait()
            pltpu.async_copy(rows_v.at[b], acc_sp.at[cidx.at[h, 1, t]],
                             ssems[b], add=True)
        return carry

    lax.fori_loop(0, NSEG // 2, body, 0)
    for i in range(LAG):
        j = NCHUNK - LAG + i
        pltpu.make_async_copy(rows_v.at[j % NB], acc_sp.at[cidx.at[0, 1, 0]],
                              ssems[j % NB]).wait()
    plsc.subcore_barrier()

    @pl.when(s < 15)
    def _():
        pltpu.sync_copy(acc_sp.at[pl.ds(row0, RCH)],
                        out_hbm.at[c, pl.ds(row0, RCH)])

    @pl.when(s == 15)
    def _():
        pltpu.sync_copy(acc_sp.at[pl.ds(row0, RLAST)],
                        out_hbm.at[c, pl.ds(row0, RLAST)])


BLK = 1000
GRID = N // BLK


def _tc_xw1_body(x_ref, deg_ref, w1td_ref, w1bu_ref, root_ref,
                 w2td_ref, w2bu_ref, out_ref, rtd_ref, rbu_ref, xroot_ref):
    i = pl.program_id(0)
    dinv_td = lax.rsqrt(deg_ref[:, 0:1] + 1.0)
    dinv_bu = lax.rsqrt(deg_ref[:, 1:2] + 1.0)
    xb = x_ref[...]
    out_ref[0] = jnp.dot(xb, w1td_ref[...],
                         preferred_element_type=jnp.float32) * dinv_td
    out_ref[1] = jnp.dot(xb, w1bu_ref[...],
                         preferred_element_type=jnp.float32) * dinv_bu

    rowsel = (lax.broadcasted_iota(jnp.int32, (B, BLK), 1) + i * BLK
              == root_ref[...]).astype(jnp.float32)
    part = jnp.dot(rowsel, xb, preferred_element_type=jnp.float32)

    @pl.when(i == 0)
    def _():
        xroot_ref[...] = part

    @pl.when(i > 0)
    def _():
        xroot_ref[...] += part

    @pl.when(i == GRID - 1)
    def _():
        rx = jnp.maximum(xroot_ref[...], 0.0)
        rtd_ref[...] = jnp.dot(rx, w2td_ref[F:, :],
                               preferred_element_type=jnp.float32)
        rbu_ref[...] = jnp.dot(rx, w2bu_ref[F:, :],
                               preferred_element_type=jnp.float32)


def _tc_xw1(x, degF, W1_td, W1_bu, root2, W2_td, W2_bu):
    return pl.pallas_call(
        _tc_xw1_body,
        grid=(GRID,),
        in_specs=[
            pl.BlockSpec((BLK, F), lambda i: (i, 0)),
            pl.BlockSpec((BLK, 2), lambda i: (i, 0)),
            pl.BlockSpec((F, F), lambda i: (0, 0)),
            pl.BlockSpec((F, F), lambda i: (0, 0)),
            pl.BlockSpec((B, 1), lambda i: (0, 0)),
            pl.BlockSpec((2 * F, F), lambda i: (0, 0)),
            pl.BlockSpec((2 * F, F), lambda i: (0, 0)),
        ],
        out_specs=(
            pl.BlockSpec((2, BLK, F), lambda i: (0, i, 0)),
            pl.BlockSpec((B, F), lambda i: (0, 0)),
            pl.BlockSpec((B, F), lambda i: (0, 0)),
        ),
        out_shape=(
            jax.ShapeDtypeStruct((2, N, F), jnp.float32),
            jax.ShapeDtypeStruct((B, F), jnp.float32),
            jax.ShapeDtypeStruct((B, F), jnp.float32),
        ),
        scratch_shapes=[pltpu.VMEM((B, F), jnp.float32)],
    )(x, degF, W1_td, W1_bu, root2, W2_td, W2_bu)


def _tc_mid_body(acc1_ref, xw1_ref, deg_ref, batch_ref, root_ref,
                 rtd_ref, rbu_ref, w2td_ref, w2bu_ref, b1td_ref, b1bu_ref,
                 xw2_ref, x2root_ref):
    i = pl.program_id(0)
    ohb = (lax.broadcasted_iota(jnp.int32, (BLK, B), 1)
           == batch_ref[...]).astype(jnp.float32)
    rowsel = (lax.broadcasted_iota(jnp.int32, (B, BLK), 1) + i * BLK
              == root_ref[...]).astype(jnp.float32)

    def one_branch(k, w2_ref, r_ref, b1_ref):
        dinv = lax.rsqrt(deg_ref[:, k:k + 1] + 1.0)
        h1 = dinv * (acc1_ref[k] + xw1_ref[k]) + b1_ref[...]
        rh = jnp.maximum(h1, 0.0)
        xw2 = (jnp.dot(rh, w2_ref[:F, :], preferred_element_type=jnp.float32)
               + jnp.dot(ohb, r_ref[...], preferred_element_type=jnp.float32)
               ) * dinv
        xw2_ref[k] = xw2
        part = jnp.dot(rowsel, h1, preferred_element_type=jnp.float32)
        return part

    ptd = one_branch(0, w2td_ref, rtd_ref, b1td_ref)
    pbu = one_branch(1, w2bu_ref, rbu_ref, b1bu_ref)

    @pl.when(i == 0)
    def _():
        x2root_ref[0] = ptd
        x2root_ref[1] = pbu

    @pl.when(i > 0)
    def _():
        x2root_ref[0] += ptd
        x2root_ref[1] += pbu


def _tc_mid(acc1, xw1, degF, batch2, root2, R_td, R_bu,
            W2_td, W2_bu, b1_td2, b1_bu2):
    return pl.pallas_call(
        _tc_mid_body,
        grid=(GRID,),
        in_specs=[
            pl.BlockSpec((2, BLK, F), lambda i: (0, i, 0)),
            pl.BlockSpec((2, BLK, F), lambda i: (0, i, 0)),
            pl.BlockSpec((BLK, 2), lambda i: (i, 0)),
            pl.BlockSpec((BLK, 1), lambda i: (i, 0)),
            pl.BlockSpec((B, 1), lambda i: (0, 0)),
            pl.BlockSpec((B, F), lambda i: (0, 0)),
            pl.BlockSpec((B, F), lambda i: (0, 0)),
            pl.BlockSpec((2 * F, F), lambda i: (0, 0)),
            pl.BlockSpec((2 * F, F), lambda i: (0, 0)),
            pl.BlockSpec((1, F), lambda i: (0, 0)),
            pl.BlockSpec((1, F), lambda i: (0, 0)),
        ],
        out_specs=(
            pl.BlockSpec((2, BLK, F), lambda i: (0, i, 0)),
            pl.BlockSpec((2, B, F), lambda i: (0, 0, 0)),
        ),
        out_shape=(
            jax.ShapeDtypeStruct((2, N, F), jnp.float32),
            jax.ShapeDtypeStruct((2, B, F), jnp.float32),
        ),
    )(acc1, xw1, degF, batch2, root2, R_td, R_bu, W2_td, W2_bu,
      b1_td2, b1_bu2)


def _tc_final_body(acc2_ref, xw2_ref, deg_ref, batch_ref,
                   b2td_ref, b2bu_ref, x2root_ref, fcw_ref, fcb_ref,
                   out_ref, std_ref, sbu_ref, cnt_ref):
    i = pl.program_id(0)
    ohb = (lax.broadcasted_iota(jnp.int32, (BLK, B), 1)
           == batch_ref[...]).astype(jnp.float32)
    dn = (((0,), (0,)), ((), ()))

    def branch_p(k, b2_ref):
        dinv = lax.rsqrt(deg_ref[:, k:k + 1] + 1.0)
        out2 = dinv * (acc2_ref[k] + xw2_ref[k]) + b2_ref[...]
        return jnp.maximum(out2, 0.0)

    ptd = branch_p(0, b2td_ref)
    pbu = branch_p(1, b2bu_ref)
    std = lax.dot_general(ohb, ptd, dn, preferred_element_type=jnp.float32)
    sbu = lax.dot_general(ohb, pbu, dn, preferred_element_type=jnp.float32)
    ones_col = jnp.ones((BLK, 1), jnp.float32)
    cnt = lax.dot_general(ohb, ones_col, dn,
                          preferred_element_type=jnp.float32)

    @pl.when(i == 0)
    def _():
        std_ref[...] = std
        sbu_ref[...] = sbu
        cnt_ref[...] = cnt

    @pl.when(i > 0)
    def _():
        std_ref[...] += std
        sbu_ref[...] += sbu
        cnt_ref[...] += cnt

    @pl.when(i == GRID - 1)
    def _():
        counts = cnt_ref[...]
        denom = jnp.maximum(counts, 1.0)
        nonempty = counts > 0.0
        p1td = std_ref[...] / denom
        p1bu = sbu_ref[...] / denom
        p2td = jnp.where(nonempty, x2root_ref[0], 0.0)
        p2bu = jnp.where(nonempty, x2root_ref[1], 0.0)
        hfc = jnp.concatenate([p1td, p2td, p1bu, p2bu], axis=1)
        logits = jnp.dot(hfc, fcw_ref[...],
                         preferred_element_type=jnp.float32) + fcb_ref[...]
        m = jnp.max(logits, axis=1, keepdims=True)
        lse = m + jnp.log(jnp.sum(jnp.exp(logits - m), axis=1, keepdims=True))
        out_ref[...] = logits - lse


def _tc_final(acc2, xw2, degF, batch2, b2_td2, b2_bu2, x2_root, fc_W, fc_b2):
    return pl.pallas_call(
        _tc_final_body,
        grid=(GRID,),
        in_specs=[
            pl.BlockSpec((2, BLK, F), lambda i: (0, i, 0)),
            pl.BlockSpec((2, BLK, F), lambda i: (0, i, 0)),
            pl.BlockSpec((BLK, 2), lambda i: (i, 0)),
            pl.BlockSpec((BLK, 1), lambda i: (i, 0)),
            pl.BlockSpec((1, F), lambda i: (0, 0)),
            pl.BlockSpec((1, F), lambda i: (0, 0)),
            pl.BlockSpec((2, B, F), lambda i: (0, 0, 0)),
            pl.BlockSpec((4 * F, 2), lambda i: (0, 0)),
            pl.BlockSpec((1, 2), lambda i: (0, 0)),
        ],
        out_specs=pl.BlockSpec((B, 2), lambda i: (0, 0)),
        out_shape=jax.ShapeDtypeStruct((B, 2), jnp.float32),
        scratch_shapes=[
            pltpu.VMEM((B, F), jnp.float32),
            pltpu.VMEM((B, F), jnp.float32),
            pltpu.VMEM((B, 1), jnp.float32),
        ],
    )(acc2, xw2, degF, batch2, b2_td2, b2_bu2, x2_root, fc_W, fc_b2)


def kernel(x, edge_index, BU_edge_index, root_index, batch,
           W1_td, b1_td, W2_td, b2_td,
           W1_bu, b1_bu, W2_bu, b2_bu,
           fc_W, fc_b):
    src2 = jnp.stack([edge_index[0], BU_edge_index[0]]
                     ).reshape(NCORE, NSUB, NSEG, SEG, K)
    dst2 = jnp.stack([edge_index[1], BU_edge_index[1]]
                     ).reshape(NCORE, NSUB, NSEG, SEG, K)
    dstD = dst2.reshape(NCORE, NSUB, EPT)
    root2 = root_index.reshape(B, 1)
    batch2 = batch.reshape(N, 1)
    b1_td2 = b1_td.reshape(1, F)
    b1_bu2 = b1_bu.reshape(1, F)
    b2_td2 = b2_td.reshape(1, F)
    b2_bu2 = b2_bu.reshape(1, F)
    fc_b2 = fc_b.reshape(1, 2)

    zeros_n = jnp.zeros((NPAD,), jnp.float32)
    zeros_rows = jnp.zeros((RLAST, F), jnp.float32)

    deg2 = _sc_degree(dstD, zeros_n)
    degF = deg2[:, :N].T

    xw1, R_td, R_bu = _tc_xw1(x, degF, W1_td, W1_bu, root2, W2_td, W2_bu)
    acc1 = _sc_scatter(xw1, src2, dst2, zeros_rows)

    xw2, x2_root = _tc_mid(acc1, xw1, degF, batch2, root2, R_td, R_bu,
                           W2_td, W2_bu, b1_td2, b1_bu2)
    acc2 = _sc_scatter(xw2, src2, dst2, zeros_rows)

    return _tc_final(acc2, xw2, degF, batch2, b2_td2, b2_bu2,
                     x2_root, fc_W, fc_b2)


# speedup vs baseline: 34.6005x; 1.0113x over previous
import functools

import jax
import jax.numpy as jnp
from jax import lax
from jax.experimental import pallas as pl
from jax.experimental.pallas import tpu as pltpu
from jax.experimental.pallas import tpu_sc as plsc

N = 10000
E = 320000
B = 64
F = 128

NCORE = 2
NSUB = 16
EPT = E // NSUB
K = 100
NCHUNK = EPT // K
SEG = 10
NSEG = NCHUNK // SEG
LANES = 16
NPAD = 10112
RC = 640
RCL = NPAD - 15 * RC

RCH = 624
RLAST = N - 15 * RCH

_mesh = functools.partial(
    plsc.VectorSubcoreMesh, core_axis_name="c", subcore_axis_name="s")


NB = 2


@functools.partial(
    pl.kernel,
    mesh=_mesh(),
    compiler_params=pltpu.CompilerParams(needs_layout_passes=False),
    out_type=jax.ShapeDtypeStruct((NCORE, NPAD), jnp.float32),
    scratch_types=[
        pltpu.VMEM((NPAD,), jnp.float32),
        pltpu.VMEM((EPT,), jnp.int32),
        pltpu.VMEM((NSUB, RC), jnp.float32),
        pltpu.VMEM((RC,), jnp.float32),
        pltpu.VMEM_SHARED((NSUB, NPAD), jnp.float32),
    ],
)
def _sc_degree(dst_hbm, zeros_hbm, out_hbm, hist, didx, rbuf, sumv, stage):
    c = lax.axis_index("c")
    s = lax.axis_index("s")

    pltpu.sync_copy(zeros_hbm, hist)
    pltpu.sync_copy(dst_hbm.at[c, s], didx)
    ones16 = jnp.full((LANES,), 1.0, jnp.float32)

    def body(i, carry):
        idx = didx[pl.ds(i * LANES, LANES)]
        plsc.addupdate_scatter(hist, [idx], ones16)
        return carry

    lax.fori_loop(0, EPT // LANES, body, 0)
    pltpu.sync_copy(hist, stage.at[s])
    plsc.subcore_barrier()

    col0 = pl.multiple_of(s * RC, 8)

    @pl.when(s < 15)
    def _():
        pltpu.sync_copy(stage.at[:, pl.ds(col0, RC)], rbuf)

    @pl.when(s == 15)
    def _():
        pltpu.sync_copy(stage.at[:, pl.ds(col0, RCL)],
                        rbuf.at[:, pl.ds(0, RCL)])

    def msum(m, carry):
        acc = rbuf[0, pl.ds(m * LANES, LANES)]
        for r in range(1, NSUB):
            acc = acc + rbuf[r, pl.ds(m * LANES, LANES)]
        sumv[pl.ds(m * LANES, LANES)] = acc
        return carry

    lax.fori_loop(0, RC // LANES, msum, 0)

    @pl.when(s < 15)
    def _():
        pltpu.sync_copy(sumv, out_hbm.at[c, pl.ds(col0, RC)])

    @pl.when(s == 15)
    def _():
        pltpu.sync_copy(sumv.at[pl.ds(0, RCL)], out_hbm.at[c, pl.ds(col0, RCL)])


LAG = 1


@functools.partial(
    pl.kernel,
    mesh=_mesh(),
    out_type=jax.ShapeDtypeStruct((NCORE, N, F), jnp.float32),
    scratch_types=[
        pltpu.VMEM((2, 2, SEG, K), jnp.int32),
        pltpu.VMEM((NB, K, F), jnp.float32),
        pltpu.VMEM_SHARED((N, F), jnp.float32),
    ] + [pltpu.SemaphoreType.DMA] * (2 * NB + 2),
)
def _sc_scatter(tab_hbm, src_hbm, dst_hbm, zeros_hbm, out_hbm,
                cidx, rows_v, acc_sp, *sems):
    gsems, ssems, isems = sems[:NB], sems[NB:2 * NB], sems[2 * NB:]
    c = lax.axis_index("c")
    s = lax.axis_index("s")
    ahead = NB - LAG

    row0 = pl.multiple_of(s * RCH, 8)

    @pl.when(s < 15)
    def _():
        pltpu.sync_copy(zeros_hbm.at[pl.ds(0, RCH)], acc_sp.at[pl.ds(row0, RCH)])

    @pl.when(s == 15)
    def _():
        pltpu.sync_copy(zeros_hbm, acc_sp.at[pl.ds(row0, RLAST)])

    pltpu.sync_copy(src_hbm.at[c, s, 0], cidx.at[0, 0])
    pltpu.sync_copy(dst_hbm.at[c, s, 0], cidx.at[0, 1])
    tab_c = tab_hbm.at[c]
    plsc.subcore_barrier()

    for t in range(ahead):
        pltpu.async_copy(tab_c.at[cidx.at[0, 0, t]], rows_v.at[t], gsems[t])

    def body(g, carry):
        for sl in range(2 * SEG):
            h, t = divmod(sl, SEG)
            j = g * (2 * SEG) + sl
            b = sl % NB
            bb = (b + NB - LAG) % NB

            @pl.when(j >= LAG)
            def _():
                pltpu.make_async_copy(rows_v.at[bb],
                                      acc_sp.at[cidx.at[0, 1, 0]],
                                      ssems[bb]).wait()

            if sl == LAG + 1:
                pltpu.async_copy(src_hbm.at[c, s, 2 * g + 1], cidx.at[1, 0],
                                 isems[1])
                pltpu.async_copy(dst_hbm.at[c, s, 2 * g + 1], cidx.at[1, 1],
                                 isems[1])
            if sl == SEG + LAG + 1:
                @pl.when(g < NSEG // 2 - 1)
                def _():
                    pltpu.async_copy(src_hbm.at[c, s, 2 * g + 2], cidx.at[0, 0],
                                     isems[0])
                    pltpu.async_copy(dst_hbm.at[c, s, 2 * g + 2], cidx.at[0, 1],
                                     isems[0])
            if sl == SEG - ahead:
                pltpu.make_async_copy(src_hbm.at[c, s, 0], cidx.at[1, 0],
                                      isems[1]).wait()
                pltpu.make_async_copy(dst_hbm.at[c, s, 0], cidx.at[1, 1],
                                      isems[1]).wait()
            if sl == 2 * SEG - ahead:
                @pl.when(g < NSEG // 2 - 1)
                def _():
                    pltpu.make_async_copy(src_hbm.at[c, s, 0], cidx.at[0, 0],
                                          isems[0]).wait()
                    pltpu.make_async_copy(dst_hbm.at[c, s, 0], cidx.at[0, 1],
                                          isems[0]).wait()

            h3, t3 = divmod((sl + ahead) % (2 * SEG), SEG)

            @pl.when(j < NCHUNK - ahead)
            def _():
                pltpu.async_copy(tab_c.at[cidx.at[h3, 0, t3]],
                                 rows_v.at[bb], gsems[bb])

            pltpu.make_async_copy(tab_c.at[cidx.at[h, 0, t]],
                                  rows_v.at[b], gsems[b]).wait()
            pltpu.async_copy(rows_v.at[b], acc_sp.at[cidx.at[h, 1, t]],
                             ssems[b], add=True)
        return carry

    lax.fori_loop(0, NSEG // 2, body, 0)
    for i in range(LAG):
        j = NCHUNK - LAG + i
        pltpu.make_async_copy(rows_v.at[j % NB], acc_sp.at[cidx.at[0, 1, 0]],
                              ssems[j % NB]).wait()
    plsc.subcore_barrier()

    @pl.when(s < 15)
    def _():
        pltpu.sync_copy(acc_sp.at[pl.ds(row0, RCH)],
                        out_hbm.at[c, pl.ds(row0, RCH)])

    @pl.when(s == 15)
    def _():
        pltpu.sync_copy(acc_sp.at[pl.ds(row0, RLAST)],
                        out_hbm.at[c, pl.ds(row0, RLAST)])


BLK = 2000
GRID = N // BLK


def _tc_xw1_body(x_ref, deg_ref, w1td_ref, w1bu_ref, root_ref,
                 w2td_ref, w2bu_ref, out_ref, rtd_ref, rbu_ref, xroot_ref):
    i = pl.program_id(0)
    dinv_td = lax.rsqrt(deg_ref[:, 0:1] + 1.0)
    dinv_bu = lax.rsqrt(deg_ref[:, 1:2] + 1.0)
    xb = x_ref[...]
    out_ref[0] = jnp.dot(xb, w1td_ref[...],
                         preferred_element_type=jnp.float32) * dinv_td
    out_ref[1] = jnp.dot(xb, w1bu_ref[...],
                         preferred_element_type=jnp.float32) * dinv_bu

    rowsel = (lax.broadcasted_iota(jnp.int32, (B, BLK), 1) + i * BLK
              == root_ref[...]).astype(jnp.float32)
    part = jnp.dot(rowsel, xb, preferred_element_type=jnp.float32)

    @pl.when(i == 0)
    def _():
        xroot_ref[...] = part

    @pl.when(i > 0)
    def _():
        xroot_ref[...] += part

    @pl.when(i == GRID - 1)
    def _():
        rx = jnp.maximum(xroot_ref[...], 0.0)
        rtd_ref[...] = jnp.dot(rx, w2td_ref[F:, :],
                               preferred_element_type=jnp.float32)
        rbu_ref[...] = jnp.dot(rx, w2bu_ref[F:, :],
                               preferred_element_type=jnp.float32)


def _tc_xw1(x, degF, W1_td, W1_bu, root2, W2_td, W2_bu):
    return pl.pallas_call(
        _tc_xw1_body,
        grid=(GRID,),
        in_specs=[
            pl.BlockSpec((BLK, F), lambda i: (i, 0)),
            pl.BlockSpec((BLK, 2), lambda i: (i, 0)),
            pl.BlockSpec((F, F), lambda i: (0, 0)),
            pl.BlockSpec((F, F), lambda i: (0, 0)),
            pl.BlockSpec((B, 1), lambda i: (0, 0)),
            pl.BlockSpec((2 * F, F), lambda i: (0, 0)),
            pl.BlockSpec((2 * F, F), lambda i: (0, 0)),
        ],
        out_specs=(
            pl.BlockSpec((2, BLK, F), lambda i: (0, i, 0)),
            pl.BlockSpec((B, F), lambda i: (0, 0)),
            pl.BlockSpec((B, F), lambda i: (0, 0)),
        ),
        out_shape=(
            jax.ShapeDtypeStruct((2, N, F), jnp.float32),
            jax.ShapeDtypeStruct((B, F), jnp.float32),
            jax.ShapeDtypeStruct((B, F), jnp.float32),
        ),
        scratch_shapes=[pltpu.VMEM((B, F), jnp.float32)],
    )(x, degF, W1_td, W1_bu, root2, W2_td, W2_bu)


def _tc_mid_body(acc1_ref, xw1_ref, deg_ref, batch_ref, root_ref,
                 rtd_ref, rbu_ref, w2td_ref, w2bu_ref, b1td_ref, b1bu_ref,
                 xw2_ref, x2root_ref):
    i = pl.program_id(0)
    ohb = (lax.broadcasted_iota(jnp.int32, (BLK, B), 1)
           == batch_ref[...]).astype(jnp.float32)
    rowsel = (lax.broadcasted_iota(jnp.int32, (B, BLK), 1) + i * BLK
              == root_ref[...]).astype(jnp.float32)

    def one_branch(k, w2_ref, r_ref, b1_ref):
        dinv = lax.rsqrt(deg_ref[:, k:k + 1] + 1.0)
        h1 = dinv * (acc1_ref[k] + xw1_ref[k]) + b1_ref[...]
        rh = jnp.maximum(h1, 0.0)
        xw2 = (jnp.dot(rh, w2_ref[:F, :], preferred_element_type=jnp.float32)
               + jnp.dot(ohb, r_ref[...], preferred_element_type=jnp.float32)
               ) * dinv
        xw2_ref[k] = xw2
        part = jnp.dot(rowsel, h1, preferred_element_type=jnp.float32)
        return part

    ptd = one_branch(0, w2td_ref, rtd_ref, b1td_ref)
    pbu = one_branch(1, w2bu_ref, rbu_ref, b1bu_ref)

    @pl.when(i == 0)
    def _():
        x2root_ref[0] = ptd
        x2root_ref[1] = pbu

    @pl.when(i > 0)
    def _():
        x2root_ref[0] += ptd
        x2root_ref[1] += pbu


def _tc_mid(acc1, xw1, degF, batch2, root2, R_td, R_bu,
            W2_td, W2_bu, b1_td2, b1_bu2):
    return pl.pallas_call(
        _tc_mid_body,
        grid=(GRID,),
        in_specs=[
            pl.BlockSpec((2, BLK, F), lambda i: (0, i, 0)),
            pl.BlockSpec((2, BLK, F), lambda i: (0, i, 0)),
            pl.BlockSpec((BLK, 2), lambda i: (i, 0)),
            pl.BlockSpec((BLK, 1), lambda i: (i, 0)),
            pl.BlockSpec((B, 1), lambda i: (0, 0)),
            pl.BlockSpec((B, F), lambda i: (0, 0)),
            pl.BlockSpec((B, F), lambda i: (0, 0)),
            pl.BlockSpec((2 * F, F), lambda i: (0, 0)),
            pl.BlockSpec((2 * F, F), lambda i: (0, 0)),
            pl.BlockSpec((1, F), lambda i: (0, 0)),
            pl.BlockSpec((1, F), lambda i: (0, 0)),
        ],
        out_specs=(
            pl.BlockSpec((2, BLK, F), lambda i: (0, i, 0)),
            pl.BlockSpec((2, B, F), lambda i: (0, 0, 0)),
        ),
        out_shape=(
            jax.ShapeDtypeStruct((2, N, F), jnp.float32),
            jax.ShapeDtypeStruct((2, B, F), jnp.float32),
        ),
    )(acc1, xw1, degF, batch2, root2, R_td, R_bu, W2_td, W2_bu,
      b1_td2, b1_bu2)


def _tc_final_body(acc2_ref, xw2_ref, deg_ref, batch_ref,
                   b2td_ref, b2bu_ref, x2root_ref, fcw_ref, fcb_ref,
                   out_ref, std_ref, sbu_ref, cnt_ref):
    i = pl.program_id(0)
    ohb = (lax.broadcasted_iota(jnp.int32, (BLK, B), 1)
           == batch_ref[...]).astype(jnp.float32)
    dn = (((0,), (0,)), ((), ()))

    def branch_p(k, b2_ref):
        dinv = lax.rsqrt(deg_ref[:, k:k + 1] + 1.0)
        out2 = dinv * (acc2_ref[k] + xw2_ref[k]) + b2_ref[...]
        return jnp.maximum(out2, 0.0)

    ptd = branch_p(0, b2td_ref)
    pbu = branch_p(1, b2bu_ref)
    std = lax.dot_general(ohb, ptd, dn, preferred_element_type=jnp.float32)
    sbu = lax.dot_general(ohb, pbu, dn, preferred_element_type=jnp.float32)
    ones_col = jnp.ones((BLK, 1), jnp.float32)
    cnt = lax.dot_general(ohb, ones_col, dn,
                          preferred_element_type=jnp.float32)

    @pl.when(i == 0)
    def _():
        std_ref[...] = std
        sbu_ref[...] = sbu
        cnt_ref[...] = cnt

    @pl.when(i > 0)
    def _():
        std_ref[...] += std
        sbu_ref[...] += sbu
        cnt_ref[...] += cnt

    @pl.when(i == GRID - 1)
    def _():
        counts = cnt_ref[...]
        denom = jnp.maximum(counts, 1.0)
        nonempty = counts > 0.0
        p1td = std_ref[...] / denom
        p1bu = sbu_ref[...] / denom
        p2td = jnp.where(nonempty, x2root_ref[0], 0.0)
        p2bu = jnp.where(nonempty, x2root_ref[1], 0.0)
        hfc = jnp.concatenate([p1td, p2td, p1bu, p2bu], axis=1)
        logits = jnp.dot(hfc, fcw_ref[...],
                         preferred_element_type=jnp.float32) + fcb_ref[...]
        m = jnp.max(logits, axis=1, keepdims=True)
        lse = m + jnp.log(jnp.sum(jnp.exp(logits - m), axis=1, keepdims=True))
        out_ref[...] = logits - lse


def _tc_final(acc2, xw2, degF, batch2, b2_td2, b2_bu2, x2_root, fc_W, fc_b2):
    return pl.pallas_call(
        _tc_final_body,
        grid=(GRID,),
        in_specs=[
            pl.BlockSpec((2, BLK, F), lambda i: (0, i, 0)),
            pl.BlockSpec((2, BLK, F), lambda i: (0, i, 0)),
            pl.BlockSpec((BLK, 2), lambda i: (i, 0)),
            pl.BlockSpec((BLK, 1), lambda i: (i, 0)),
            pl.BlockSpec((1, F), lambda i: (0, 0)),
            pl.BlockSpec((1, F), lambda i: (0, 0)),
            pl.BlockSpec((2, B, F), lambda i: (0, 0, 0)),
            pl.BlockSpec((4 * F, 2), lambda i: (0, 0)),
            pl.BlockSpec((1, 2), lambda i: (0, 0)),
        ],
        out_specs=pl.BlockSpec((B, 2), lambda i: (0, 0)),
        out_shape=jax.ShapeDtypeStruct((B, 2), jnp.float32),
        scratch_shapes=[
            pltpu.VMEM((B, F), jnp.float32),
            pltpu.VMEM((B, F), jnp.float32),
            pltpu.VMEM((B, 1), jnp.float32),
        ],
    )(acc2, xw2, degF, batch2, b2_td2, b2_bu2, x2_root, fc_W, fc_b2)


def kernel(x, edge_index, BU_edge_index, root_index, batch,
           W1_td, b1_td, W2_td, b2_td,
           W1_bu, b1_bu, W2_bu, b2_bu,
           fc_W, fc_b):
    src2 = jnp.stack([edge_index[0], BU_edge_index[0]]
                     ).reshape(NCORE, NSUB, NSEG, SEG, K)
    dst2 = jnp.stack([edge_index[1], BU_edge_index[1]]
                     ).reshape(NCORE, NSUB, NSEG, SEG, K)
    dstD = dst2.reshape(NCORE, NSUB, EPT)
    root2 = root_index.reshape(B, 1)
    batch2 = batch.reshape(N, 1)
    b1_td2 = b1_td.reshape(1, F)
    b1_bu2 = b1_bu.reshape(1, F)
    b2_td2 = b2_td.reshape(1, F)
    b2_bu2 = b2_bu.reshape(1, F)
    fc_b2 = fc_b.reshape(1, 2)

    zeros_n = jnp.zeros((NPAD,), jnp.float32)
    zeros_rows = jnp.zeros((RLAST, F), jnp.float32)

    deg2 = _sc_degree(dstD, zeros_n)
    degF = deg2[:, :N].T

    xw1, R_td, R_bu = _tc_xw1(x, degF, W1_td, W1_bu, root2, W2_td, W2_bu)
    acc1 = _sc_scatter(xw1, src2, dst2, zeros_rows)

    xw2, x2_root = _tc_mid(acc1, xw1, degF, batch2, root2, R_td, R_bu,
                           W2_td, W2_bu, b1_td2, b1_bu2)
    acc2 = _sc_scatter(xw2, src2, dst2, zeros_rows)

    return _tc_final(acc2, xw2, degF, batch2, b2_td2, b2_bu2,
                     x2_root, fc_W, fc_b2)
